# baseline (speedup 1.0000x reference)
import functools

import jax
import jax.numpy as jnp
from jax import lax
from jax.experimental import pallas as pl
from jax.experimental.pallas import tpu as pltpu

N_FM = 50000
N_TP = 8192
N_SM = 20000
H = 128

P_FM = 50176
P_TP = 8320
P_SM = 20096


def _prep_body(wpe, pvol, wr_qoq, wr_cp, wr_rev, bl_qoq, bl_cp, bl_rev,
               pe_out, wr_out, bl_out):
    pe_out[...] = wpe[...] * pvol[...]
    wr_out[...] = wr_qoq[...] + wr_cp[...] + wr_rev[...]
    bl_out[...] = bl_qoq[...] + bl_cp[...] + bl_rev[...]


def _prep(wpe, pvol, wr_qoq, wr_cp, wr_rev, bl_qoq, bl_cp, bl_rev):
    return pl.pallas_call(
        _prep_body,
        out_shape=(
            jax.ShapeDtypeStruct((N_TP, 32), jnp.float32),
            jax.ShapeDtypeStruct((H, H), jnp.float32),
            jax.ShapeDtypeStruct((1, H), jnp.float32),
        ),
    )(wpe, pvol, wr_qoq, wr_cp, wr_rev, bl_qoq, bl_cp, bl_rev)


def _ln_relu(o, g, b):
    mu = jnp.mean(o, axis=1, keepdims=True)
    d = o - mu
    v = jnp.mean(d * d, axis=1, keepdims=True)
    y = d * lax.rsqrt(v + 1e-5) * g + b
    return jnp.maximum(y, 0.0)


def _inv_counts(cnt_ref):
    c = jnp.sum(cnt_ref[0] + cnt_ref[1], axis=1, keepdims=True)
    return 1.0 / jnp.maximum(c, 1.0)


def _chunk_matmul(s_ref, inv, wl_ref):
    k = s_ref.shape[1]
    out = None
    for c in range(k):
        agg = (s_ref[0, c] + s_ref[1, c]) * inv
        part = jax.lax.dot_general(
            agg, wl_ref[:, 32 * c:32 * c + 32],
            (((1,), (1,)), ((), ())),
            preferred_element_type=jnp.float32)
        out = part if out is None else out + part
    return out


def _fm_body(x_ref, sq_ref, cq_ref, sc_ref, cc_ref, sr_ref, cr_ref,
             wlq_ref, wlc_ref, wlr_ref, wr_ref, bl_ref, g_ref, b_ref,
             out_ref, nblk):
    i = pl.program_id(0)
    o = _chunk_matmul(sq_ref, _inv_counts(cq_ref), wlq_ref)
    o += _chunk_matmul(sc_ref, _inv_counts(cc_ref), wlc_ref)
    o += _chunk_matmul(sr_ref, _inv_counts(cr_ref), wlr_ref)
    o += jax.lax.dot_general(x_ref[...], wr_ref[...],
                             (((1,), (1,)), ((), ())),
                             preferred_element_type=jnp.float32)
    o = (o + bl_ref[...]) / 3.0
    y = _ln_relu(o, g_ref[...], b_ref[...])
    s = jnp.sum(y, axis=0, keepdims=True)

    @pl.when(i == 0)
    def _():
        out_ref[...] = jnp.zeros_like(out_ref)
    out_ref[...] += s
    @pl.when(i == nblk - 1)
    def _():
        out_ref[...] = out_ref[...] * (1.0 / N_FM)


def _fm_mean(x_fm, sq, cq, scp, ccp, srv, crv, p, wr_sum, bl_sum):
    blk, nblk = 400, 125
    grid = (nblk,)
    full = lambda *s: pl.BlockSpec(s, lambda i: (0,) * len(s))
    return pl.pallas_call(
        functools.partial(_fm_body, nblk=nblk),
        grid=grid,
        in_specs=[
            pl.BlockSpec((blk, H), lambda i: (i, 0)),
            pl.BlockSpec((2, 4, blk, 32), lambda i: (0, 0, i, 0)),
            pl.BlockSpec((2, blk, 16), lambda i: (0, i, 0)),
            pl.BlockSpec((2, 1, blk, 32), lambda i: (0, 0, i, 0)),
            pl.BlockSpec((2, blk, 16), lambda i: (0, i, 0)),
            pl.BlockSpec((2, 1, blk, 32), lambda i: (0, 0, i, 0)),
            pl.BlockSpec((2, blk, 16), lambda i: (0, i, 0)),
            full(H, H), full(H, 32), full(H, 32), full(H, H),
            full(1, H), full(1, H), full(1, H),
        ],
        out_specs=pl.BlockSpec((1, H), lambda i: (0, 0)),
        out_shape=jax.ShapeDtypeStruct((1, H), jnp.float32),
    )(x_fm, sq, cq, scp, ccp, srv, crv,
      p["Wl_qoq"], p["Wl_cp"], p["Wl_rev"], wr_sum, bl_sum,
      p["g_fm"].reshape(1, H), p["b_fm"].reshape(1, H))


def _tp_body(pe_ref, sb_ref, cb_ref, wlb_ref, wrb_ref, bl_ref, g_ref, b_ref,
             out_ref, nblk):
    i = pl.program_id(0)
    o = _chunk_matmul(sb_ref, _inv_counts(cb_ref), wlb_ref)
    o += jax.lax.dot_general(pe_ref[...], wrb_ref[...],
                             (((1,), (1,)), ((), ())),
                             preferred_element_type=jnp.float32)
    o += bl_ref[...]
    y = _ln_relu(o, g_ref[...], b_ref[...])
    s = jnp.sum(y, axis=0, keepdims=True)

    @pl.when(i == 0)
    def _():
        out_ref[...] = jnp.zeros_like(out_ref)
    out_ref[...] += s
    @pl.when(i == nblk - 1)
    def _():
        out_ref[...] = out_ref[...] * (1.0 / N_TP)


def _tp_mean(pe, sb, cb, p):
    blk, nblk = 512, 16
    full = lambda *s: pl.BlockSpec(s, lambda i: (0,) * len(s))
    return pl.pallas_call(
        functools.partial(_tp_body, nblk=nblk),
        grid=(nblk,),
        in_specs=[
            pl.BlockSpec((blk, 32), lambda i: (i, 0)),
            pl.BlockSpec((2, 4, blk, 32), lambda i: (0, 0, i, 0)),
            pl.BlockSpec((2, blk, 16), lambda i: (0, i, 0)),
            full(H, H), full(H, 32), full(1, H), full(1, H), full(1, H),
        ],
        out_specs=pl.BlockSpec((1, H), lambda i: (0, 0)),
        out_shape=jax.ShapeDtypeStruct((1, H), jnp.float32),
    )(pe, sb, cb, p["Wl_bel"], p["Wr_bel"], p["bl_bel"].reshape(1, H),
      p["g_tp"].reshape(1, H), p["b_tp"].reshape(1, H))


def _sm_body(x_ref, sd_ref, cd_ref, wld_ref, wrd_ref, bl_ref, g_ref, b_ref,
             out_ref, nblk):
    i = pl.program_id(0)
    o = _chunk_matmul(sd_ref, _inv_counts(cd_ref), wld_ref)
    o += jax.lax.dot_general(x_ref[...], wrd_ref[...],
                             (((1,), (1,)), ((), ())),
                             preferred_element_type=jnp.float32)
    o += bl_ref[...]
    y = _ln_relu(o, g_ref[...], b_ref[...])
    s = jnp.sum(y, axis=0, keepdims=True)

    @pl.when(i == 0)
    def _():
        out_ref[...] = jnp.zeros_like(out_ref)
    out_ref[...] += s
    @pl.when(i == nblk - 1)
    def _():
        out_ref[...] = out_ref[...] * (1.0 / N_SM)


def _sm_mean(x_sm, sd, cd, p):
    blk, nblk = 400, 50
    full = lambda *s: pl.BlockSpec(s, lambda i: (0,) * len(s))
    return pl.pallas_call(
        functools.partial(_sm_body, nblk=nblk),
        grid=(nblk,),
        in_specs=[
            pl.BlockSpec((blk, H), lambda i: (i, 0)),
            pl.BlockSpec((2, 1, blk, 32), lambda i: (0, 0, i, 0)),
            pl.BlockSpec((2, blk, 16), lambda i: (0, i, 0)),
            full(H, 32), full(H, H), full(1, H), full(1, H), full(1, H),
        ],
        out_specs=pl.BlockSpec((1, H), lambda i: (0, 0)),
        out_shape=jax.ShapeDtypeStruct((1, H), jnp.float32),
    )(x_sm, sd, cd, p["Wl_cd"], p["Wr_cd"], p["bl_cd"].reshape(1, H),
      p["g_sm"].reshape(1, H), p["b_sm"].reshape(1, H))


def _head_body(fm_ref, tp_ref, sm_ref, gf_ref, w1_ref, b1_ref, w2_ref,
               b2_ref, out_ref):
    h = jnp.concatenate(
        [fm_ref[...], tp_ref[...], sm_ref[...], gf_ref[...]], axis=1)
    h1 = jax.lax.dot_general(h, w1_ref[...], (((1,), (1,)), ((), ())),
                             preferred_element_type=jnp.float32)
    h1 = jnp.maximum(h1 + b1_ref[...], 0.0)
    out_ref[0, 0] = jnp.sum(h1 * w2_ref[...]) + b2_ref[0, 0]


def _head(fm, tp, sm, gf, p):
    return pl.pallas_call(
        _head_body,
        in_specs=[pl.BlockSpec(memory_space=pltpu.VMEM)] * 7
        + [pl.BlockSpec(memory_space=pltpu.SMEM)],
        out_specs=pl.BlockSpec(memory_space=pltpu.SMEM),
        out_shape=jax.ShapeDtypeStruct((1, 1), jnp.float32),
    )(fm, tp, sm, gf, p["W1"], p["b1"].reshape(1, 64), p["W2"],
      p["b2"].reshape(1, 1))


def _agg_layout(src_x, e, n_pad, k):
    msg = jnp.take(src_x, e[0], axis=0)
    s = jax.ops.segment_sum(msg, e[1], num_segments=n_pad)
    c = jax.ops.segment_sum(jnp.ones((e.shape[1],), jnp.float32), e[1],
                            num_segments=n_pad)
    s = s.reshape(n_pad, k, 32).transpose(1, 0, 2)[None]
    s = jnp.concatenate([s, jnp.zeros_like(s)], axis=0)
    c16 = jnp.zeros((2, n_pad, 16), jnp.float32).at[0, :, 0].set(c)
    return s, c16


def kernel(x_fm, x_sm, gf, period_vol, edge_qoq, edge_bel, edge_cp, edge_cd,
           edge_rev, params):
    p = params
    pe, wr_sum, bl_sum = _prep(
        p["W_pe"], period_vol, p["Wr_qoq"], p["Wr_cp"], p["Wr_rev"],
        p["bl_qoq"].reshape(1, H), p["bl_cp"].reshape(1, H),
        p["bl_rev"].reshape(1, H))

    sq, cq = _agg_layout(x_fm, edge_qoq, P_FM, 4)
    sb, cb = _agg_layout(x_fm, edge_bel, P_TP, 4)
    scp, ccp = _agg_layout(pe, edge_cp, P_FM, 1)
    sd, cd = _agg_layout(pe, edge_cd, P_SM, 1)
    srv, crv = _agg_layout(pe, edge_rev, P_FM, 1)

    fm = _fm_mean(x_fm, sq, cq, scp, ccp, srv, crv, p, wr_sum, bl_sum)
    tp = _tp_mean(pe, sb, cb, p)
    sm = _sm_mean(x_sm, sd, cd, p)
    out = _head(fm, tp, sm, gf, p)
    return out.reshape(())


# speedup vs baseline: 5.2635x; 5.2635x over previous
import functools

import jax
import jax.numpy as jnp
from jax import lax
from jax.experimental import pallas as pl
from jax.experimental.pallas import tpu as pltpu
from jax.experimental.pallas import tpu_sc as plsc

N_FM = 50000
N_TP = 8192
N_SM = 20000
H = 128

P_FM = 50176
P_TP = 8320
P_SM = 20096


def _prep_body(wpe, pvol, wr_qoq, wr_cp, wr_rev, bl_qoq, bl_cp, bl_rev,
               pe_out, wr_out, bl_out):
    pe_out[...] = wpe[...] * pvol[...]
    wr_out[...] = wr_qoq[...] + wr_cp[...] + wr_rev[...]
    bl_out[...] = bl_qoq[...] + bl_cp[...] + bl_rev[...]


def _prep(wpe, pvol, wr_qoq, wr_cp, wr_rev, bl_qoq, bl_cp, bl_rev):
    return pl.pallas_call(
        _prep_body,
        out_shape=(
            jax.ShapeDtypeStruct((N_TP, 32), jnp.float32),
            jax.ShapeDtypeStruct((H, H), jnp.float32),
            jax.ShapeDtypeStruct((1, H), jnp.float32),
        ),
    )(wpe, pvol, wr_qoq, wr_cp, wr_rev, bl_qoq, bl_cp, bl_rev)


def _ln_relu(o, g, b):
    mu = jnp.mean(o, axis=1, keepdims=True)
    d = o - mu
    v = jnp.mean(d * d, axis=1, keepdims=True)
    y = d * lax.rsqrt(v + 1e-5) * g + b
    return jnp.maximum(y, 0.0)


def _inv_counts(cnt_ref):
    c = jnp.sum(cnt_ref[0] + cnt_ref[1], axis=1, keepdims=True)
    return 1.0 / jnp.maximum(c, 1.0)


def _chunk_matmul(s_ref, inv, wl_ref):
    k = s_ref.shape[1]
    out = None
    for c in range(k):
        agg = (s_ref[0, c] + s_ref[1, c]) * inv
        part = jax.lax.dot_general(
            agg, wl_ref[:, 32 * c:32 * c + 32],
            (((1,), (1,)), ((), ())),
            preferred_element_type=jnp.float32)
        out = part if out is None else out + part
    return out


def _fm_body(x_ref, sq_ref, cq_ref, sc_ref, cc_ref, sr_ref, cr_ref,
             wlq_ref, wlc_ref, wlr_ref, wr_ref, bl_ref, g_ref, b_ref,
             out_ref, nblk):
    i = pl.program_id(0)
    o = _chunk_matmul(sq_ref, _inv_counts(cq_ref), wlq_ref)
    o += _chunk_matmul(sc_ref, _inv_counts(cc_ref), wlc_ref)
    o += _chunk_matmul(sr_ref, _inv_counts(cr_ref), wlr_ref)
    o += jax.lax.dot_general(x_ref[...], wr_ref[...],
                             (((1,), (1,)), ((), ())),
                             preferred_element_type=jnp.float32)
    o = (o + bl_ref[...]) / 3.0
    y = _ln_relu(o, g_ref[...], b_ref[...])
    s = jnp.sum(y, axis=0, keepdims=True)

    @pl.when(i == 0)
    def _():
        out_ref[...] = jnp.zeros_like(out_ref)
    out_ref[...] += s
    @pl.when(i == nblk - 1)
    def _():
        out_ref[...] = out_ref[...] * (1.0 / N_FM)


def _fm_mean(x_fm, sq, cq, scp, ccp, srv, crv, p, wr_sum, bl_sum):
    blk, nblk = 400, 125
    grid = (nblk,)
    full = lambda *s: pl.BlockSpec(s, lambda i: (0,) * len(s))
    return pl.pallas_call(
        functools.partial(_fm_body, nblk=nblk),
        grid=grid,
        in_specs=[
            pl.BlockSpec((blk, H), lambda i: (i, 0)),
            pl.BlockSpec((2, 4, blk, 32), lambda i: (0, 0, i, 0)),
            pl.BlockSpec((2, blk, 16), lambda i: (0, i, 0)),
            pl.BlockSpec((2, 1, blk, 32), lambda i: (0, 0, i, 0)),
            pl.BlockSpec((2, blk, 16), lambda i: (0, i, 0)),
            pl.BlockSpec((2, 1, blk, 32), lambda i: (0, 0, i, 0)),
            pl.BlockSpec((2, blk, 16), lambda i: (0, i, 0)),
            full(H, H), full(H, 32), full(H, 32), full(H, H),
            full(1, H), full(1, H), full(1, H),
        ],
        out_specs=pl.BlockSpec((1, H), lambda i: (0, 0)),
        out_shape=jax.ShapeDtypeStruct((1, H), jnp.float32),
    )(x_fm, sq, cq, scp, ccp, srv, crv,
      p["Wl_qoq"], p["Wl_cp"], p["Wl_rev"], wr_sum, bl_sum,
      p["g_fm"].reshape(1, H), p["b_fm"].reshape(1, H))


def _tp_body(pe_ref, sb_ref, cb_ref, wlb_ref, wrb_ref, bl_ref, g_ref, b_ref,
             out_ref, nblk):
    i = pl.program_id(0)
    o = _chunk_matmul(sb_ref, _inv_counts(cb_ref), wlb_ref)
    o += jax.lax.dot_general(pe_ref[...], wrb_ref[...],
                             (((1,), (1,)), ((), ())),
                             preferred_element_type=jnp.float32)
    o += bl_ref[...]
    y = _ln_relu(o, g_ref[...], b_ref[...])
    s = jnp.sum(y, axis=0, keepdims=True)

    @pl.when(i == 0)
    def _():
        out_ref[...] = jnp.zeros_like(out_ref)
    out_ref[...] += s
    @pl.when(i == nblk - 1)
    def _():
        out_ref[...] = out_ref[...] * (1.0 / N_TP)


def _tp_mean(pe, sb, cb, p):
    blk, nblk = 512, 16
    full = lambda *s: pl.BlockSpec(s, lambda i: (0,) * len(s))
    return pl.pallas_call(
        functools.partial(_tp_body, nblk=nblk),
        grid=(nblk,),
        in_specs=[
            pl.BlockSpec((blk, 32), lambda i: (i, 0)),
            pl.BlockSpec((2, 4, blk, 32), lambda i: (0, 0, i, 0)),
            pl.BlockSpec((2, blk, 16), lambda i: (0, i, 0)),
            full(H, H), full(H, 32), full(1, H), full(1, H), full(1, H),
        ],
        out_specs=pl.BlockSpec((1, H), lambda i: (0, 0)),
        out_shape=jax.ShapeDtypeStruct((1, H), jnp.float32),
    )(pe, sb, cb, p["Wl_bel"], p["Wr_bel"], p["bl_bel"].reshape(1, H),
      p["g_tp"].reshape(1, H), p["b_tp"].reshape(1, H))


def _sm_body(x_ref, sd_ref, cd_ref, wld_ref, wrd_ref, bl_ref, g_ref, b_ref,
             out_ref, nblk):
    i = pl.program_id(0)
    o = _chunk_matmul(sd_ref, _inv_counts(cd_ref), wld_ref)
    o += jax.lax.dot_general(x_ref[...], wrd_ref[...],
                             (((1,), (1,)), ((), ())),
                             preferred_element_type=jnp.float32)
    o += bl_ref[...]
    y = _ln_relu(o, g_ref[...], b_ref[...])
    s = jnp.sum(y, axis=0, keepdims=True)

    @pl.when(i == 0)
    def _():
        out_ref[...] = jnp.zeros_like(out_ref)
    out_ref[...] += s
    @pl.when(i == nblk - 1)
    def _():
        out_ref[...] = out_ref[...] * (1.0 / N_SM)


def _sm_mean(x_sm, sd, cd, p):
    blk, nblk = 400, 50
    full = lambda *s: pl.BlockSpec(s, lambda i: (0,) * len(s))
    return pl.pallas_call(
        functools.partial(_sm_body, nblk=nblk),
        grid=(nblk,),
        in_specs=[
            pl.BlockSpec((blk, H), lambda i: (i, 0)),
            pl.BlockSpec((2, 1, blk, 32), lambda i: (0, 0, i, 0)),
            pl.BlockSpec((2, blk, 16), lambda i: (0, i, 0)),
            full(H, 32), full(H, H), full(1, H), full(1, H), full(1, H),
        ],
        out_specs=pl.BlockSpec((1, H), lambda i: (0, 0)),
        out_shape=jax.ShapeDtypeStruct((1, H), jnp.float32),
    )(x_sm, sd, cd, p["Wl_cd"], p["Wr_cd"], p["bl_cd"].reshape(1, H),
      p["g_sm"].reshape(1, H), p["b_sm"].reshape(1, H))


def _head_body(fm_ref, tp_ref, sm_ref, gf_ref, w1_ref, b1_ref, w2_ref,
               b2_ref, out_ref):
    h = jnp.concatenate(
        [fm_ref[...], tp_ref[...], sm_ref[...], gf_ref[...]], axis=1)
    h1 = jax.lax.dot_general(h, w1_ref[...], (((1,), (1,)), ((), ())),
                             preferred_element_type=jnp.float32)
    h1 = jnp.maximum(h1 + b1_ref[...], 0.0)
    out_ref[0, 0] = jnp.sum(h1 * w2_ref[...]) + b2_ref[0, 0]


def _head(fm, tp, sm, gf, p):
    return pl.pallas_call(
        _head_body,
        in_specs=[pl.BlockSpec(memory_space=pltpu.VMEM)] * 7
        + [pl.BlockSpec(memory_space=pltpu.SMEM)],
        out_specs=pl.BlockSpec(memory_space=pltpu.SMEM),
        out_shape=jax.ShapeDtypeStruct((1, 1), jnp.float32),
    )(fm, tp, sm, gf, p["W1"], p["b1"].reshape(1, 64), p["W2"],
      p["b2"].reshape(1, 1))


_SC_PARAMS = pltpu.CompilerParams(use_tc_tiling_on_sc=False)


@functools.lru_cache(maxsize=None)
def _sc_mesh():
    return plsc.VectorSubcoreMesh(core_axis_name="c", subcore_axis_name="s")

_CFG = {
    "qoq": (614400, P_FM, 4, 10),
    "bel": (401408, P_TP, 4, 14),
    "cp": (401408, P_FM, 1, 14),
    "cd": (204800, P_SM, 1, 10),
    "rev": (401408, P_FM, 1, 14),
}


def _zero_vmem(ref, rows, width):
    z = jnp.zeros((16,), jnp.float32)

    def zb(i, _):
        for w in range(width // 16):
            ref[i, pl.ds(w * 16, 16)] = z
        return _

    lax.fori_loop(0, rows, zb, None)


@functools.lru_cache(maxsize=None)
def _make_agg(e_pad, n_pad, k, g):
    ns = e_pad // 32 // 128
    nch = ns // g
    rpt = n_pad // 16
    zr = rpt // 8

    @functools.partial(
        pl.kernel, mesh=_sc_mesh(), compiler_params=_SC_PARAMS,
        out_type=jax.ShapeDtypeStruct((2, k, n_pad, 32), jnp.float32),
        scratch_types=[
            pltpu.VMEM((g, 128), jnp.int32),
            pltpu.VMEM((g, 128), jnp.int32),
            pltpu.VMEM((128, 32), jnp.float32),
            pltpu.VMEM((128, 32), jnp.float32),
            pltpu.VMEM((zr, 32), jnp.float32),
            pltpu.VMEM_SHARED((n_pad, 32), jnp.float32),
            pltpu.SemaphoreType.DMA,
            pltpu.SemaphoreType.DMA,
        ],
    )
    def agg(src3d, dst3d, *rest):
        tables = rest[:k]
        out = rest[k]
        src_c, dst_c, bufa, bufb, zv, acc, sema, semb = rest[k + 1:]
        sc = lax.axis_index("c")
        tl = lax.axis_index("s")
        wid = sc * 16 + tl
        base = tl * rpt

        _zero_vmem(zv, zr, 32)
        for z in range(8):
            pltpu.sync_copy(zv, acc.at[pl.ds(base + z * zr, zr)])

        for c in range(k):
            plsc.subcore_barrier()
            tab = tables[c]

            def chunk(q, _):
                pltpu.sync_copy(src3d.at[wid, pl.ds(q * g, g)], src_c)
                pltpu.sync_copy(dst3d.at[wid, pl.ds(q * g, g)], dst_c)

                def body(t, _):
                    j0 = t * 2
                    cpa = pltpu.async_copy(tab.at[src_c.at[j0]], bufa, sema)
                    cpb = pltpu.async_copy(tab.at[src_c.at[j0 + 1]], bufb,
                                           semb)
                    cpa.wait()
                    pltpu.sync_copy(bufa, acc.at[dst_c.at[j0]], add=True)
                    cpb.wait()
                    pltpu.sync_copy(bufb, acc.at[dst_c.at[j0 + 1]], add=True)
                    return _

                lax.fori_loop(0, g // 2, body, None)
                return _

            lax.fori_loop(0, nch, chunk, None)
            plsc.subcore_barrier()
            pltpu.sync_copy(acc.at[pl.ds(base, rpt)],
                            out.at[sc, c, pl.ds(base, rpt)])
            if c + 1 < k:
                for z in range(8):
                    pltpu.sync_copy(zv, acc.at[pl.ds(base + z * zr, zr)])

    return agg


@functools.lru_cache(maxsize=None)
def _make_counts(cfgs):
    max_np = max(c[1] for c in cfgs)
    max_g = max(c[2] for c in cfgs)
    rpt_max = max_np // 16
    zr = rpt_max // 8

    @functools.partial(
        pl.kernel, mesh=_sc_mesh(), compiler_params=_SC_PARAMS,
        out_type=tuple(jax.ShapeDtypeStruct((2, c[1], 16), jnp.float32)
                       for c in cfgs),
        scratch_types=[
            pltpu.VMEM((max_g, 128), jnp.int32),
            pltpu.VMEM((128, 16), jnp.float32),
            pltpu.VMEM((zr, 16), jnp.float32),
            pltpu.VMEM_SHARED((max_np, 16), jnp.float32),
            pltpu.SemaphoreType.DMA,
            pltpu.SemaphoreType.DMA,
        ],
    )
    def counts(*args):
        n = len(cfgs)
        dsts = args[:n]
        outs = args[n:2 * n]
        dst_c, ones, zv, acc, sema, semb = args[2 * n:]
        sc = lax.axis_index("c")
        tl = lax.axis_index("s")
        wid = sc * 16 + tl

        _zero_vmem(zv, zr, 16)
        one = jnp.full((16,), 1.0 / 16.0, jnp.float32)

        def ob(i, _):
            ones[i, pl.ds(0, 16)] = one
            return _

        lax.fori_loop(0, 128, ob, None)

        for r, (e_pad, np_, g) in enumerate(cfgs):
            ns = e_pad // 32 // 128
            nch = ns // g
            rpt = np_ // 16
            for z in range(8):
                pltpu.sync_copy(zv, acc.at[pl.ds(tl * rpt_max + z * zr, zr)])
            plsc.subcore_barrier()

            def chunk(q, _):
                pltpu.sync_copy(dsts[r].at[wid, pl.ds(q * g, g)],
                                dst_c.at[pl.ds(0, g)])

                def body(t, _):
                    j0 = t * 2
                    ca = pltpu.async_copy(ones, acc.at[dst_c.at[j0]], sema,
                                          add=True)
                    cb = pltpu.async_copy(ones, acc.at[dst_c.at[j0 + 1]],
                                          semb, add=True)
                    ca.wait()
                    cb.wait()
                    return _

                lax.fori_loop(0, g // 2, body, None)
                return _

            lax.fori_loop(0, nch, chunk, None)
            plsc.subcore_barrier()
            pltpu.sync_copy(acc.at[pl.ds(tl * rpt, rpt)],
                            outs[r].at[sc, pl.ds(tl * rpt, rpt)])
            if r + 1 < n:
                plsc.subcore_barrier()

    return counts


def _pad_edges(e, e_pad, n_dst):
    pad = e_pad - e.shape[1]
    src = jnp.concatenate([e[0], jnp.zeros((pad,), jnp.int32)])
    dst = jnp.concatenate([e[1], jnp.full((pad,), n_dst, jnp.int32)])
    return src.reshape(32, -1, 128), dst.reshape(32, -1, 128)


def kernel(x_fm, x_sm, gf, period_vol, edge_qoq, edge_bel, edge_cp, edge_cd,
           edge_rev, params):
    p = params
    pe, wr_sum, bl_sum = _prep(
        p["W_pe"], period_vol, p["Wr_qoq"], p["Wr_cp"], p["Wr_rev"],
        p["bl_qoq"].reshape(1, H), p["bl_cp"].reshape(1, H),
        p["bl_rev"].reshape(1, H))

    xc = tuple(x_fm[:, 32 * c:32 * (c + 1)] for c in range(4))
    edges = {"qoq": edge_qoq, "bel": edge_bel, "cp": edge_cp,
             "cd": edge_cd, "rev": edge_rev}
    ndst = {"qoq": N_FM, "bel": N_TP, "cp": N_FM, "cd": N_SM, "rev": N_FM}
    srcs, dsts = {}, {}
    for r, (e_pad, n_pad, k, g) in _CFG.items():
        srcs[r], dsts[r] = _pad_edges(edges[r], e_pad, ndst[r])

    rels = ["qoq", "bel", "cp", "cd", "rev"]
    cnt_cfg = tuple((_CFG[r][0], _CFG[r][1], _CFG[r][3]) for r in rels)
    cq, cb, ccp, cd, crv = _make_counts(cnt_cfg)(*[dsts[r] for r in rels])

    sq = _make_agg(*_CFG["qoq"])(srcs["qoq"], dsts["qoq"], *xc)
    sb = _make_agg(*_CFG["bel"])(srcs["bel"], dsts["bel"], *xc)
    scp = _make_agg(*_CFG["cp"])(srcs["cp"], dsts["cp"], pe)
    sd = _make_agg(*_CFG["cd"])(srcs["cd"], dsts["cd"], pe)
    srv = _make_agg(*_CFG["rev"])(srcs["rev"], dsts["rev"], pe)

    fm = _fm_mean(x_fm, sq, cq, scp, ccp, srv, crv, p, wr_sum, bl_sum)
    tp = _tp_mean(pe, sb, cb, p)
    sm = _sm_mean(x_sm, sd, cd, p)
    out = _head(fm, tp, sm, gf, p)
    return out.reshape(())


# speedup vs baseline: 5.8861x; 1.1183x over previous
import functools

import jax
import jax.numpy as jnp
from jax import lax
from jax.experimental import pallas as pl
from jax.experimental.pallas import tpu as pltpu
from jax.experimental.pallas import tpu_sc as plsc

N_FM = 50000
N_TP = 8192
N_SM = 20000
H = 128

P_FM = 50176
P_TP = 8320
P_SM = 20096


def _prep_body(wpe, pvol, wr_qoq, wr_cp, wr_rev, bl_qoq, bl_cp, bl_rev,
               pe_out, wr_out, bl_out):
    pe_out[...] = wpe[...] * pvol[...]
    wr_out[...] = wr_qoq[...] + wr_cp[...] + wr_rev[...]
    bl_out[...] = bl_qoq[...] + bl_cp[...] + bl_rev[...]


def _prep(wpe, pvol, wr_qoq, wr_cp, wr_rev, bl_qoq, bl_cp, bl_rev):
    return pl.pallas_call(
        _prep_body,
        out_shape=(
            jax.ShapeDtypeStruct((N_TP, 32), jnp.float32),
            jax.ShapeDtypeStruct((H, H), jnp.float32),
            jax.ShapeDtypeStruct((1, H), jnp.float32),
        ),
    )(wpe, pvol, wr_qoq, wr_cp, wr_rev, bl_qoq, bl_cp, bl_rev)


def _ln_relu(o, g, b):
    mu = jnp.mean(o, axis=1, keepdims=True)
    d = o - mu
    v = jnp.mean(d * d, axis=1, keepdims=True)
    y = d * lax.rsqrt(v + 1e-5) * g + b
    return jnp.maximum(y, 0.0)


def _inv_counts(cnt_ref):
    c = jnp.sum(cnt_ref[0] + cnt_ref[1], axis=1, keepdims=True)
    return 1.0 / jnp.maximum(c, 1.0)


def _chunk_matmul(s_ref, inv, wl_ref):
    k = s_ref.shape[1]
    out = None
    for c in range(k):
        agg = (s_ref[0, c] + s_ref[1, c]) * inv
        part = jax.lax.dot_general(
            agg, wl_ref[:, 32 * c:32 * c + 32],
            (((1,), (1,)), ((), ())),
            preferred_element_type=jnp.float32)
        out = part if out is None else out + part
    return out


def _fm_body(x_ref, sq_ref, cq_ref, sc_ref, cc_ref, sr_ref, cr_ref,
             wlq_ref, wlc_ref, wlr_ref, wr_ref, bl_ref, g_ref, b_ref,
             out_ref, nblk):
    i = pl.program_id(0)
    o = _chunk_matmul(sq_ref, _inv_counts(cq_ref), wlq_ref)
    o += _chunk_matmul(sc_ref, _inv_counts(cc_ref), wlc_ref)
    o += _chunk_matmul(sr_ref, _inv_counts(cr_ref), wlr_ref)
    o += jax.lax.dot_general(x_ref[...], wr_ref[...],
                             (((1,), (1,)), ((), ())),
                             preferred_element_type=jnp.float32)
    o = (o + bl_ref[...]) / 3.0
    y = _ln_relu(o, g_ref[...], b_ref[...])
    s = jnp.sum(y, axis=0, keepdims=True)

    @pl.when(i == 0)
    def _():
        out_ref[...] = jnp.zeros_like(out_ref)
    out_ref[...] += s
    @pl.when(i == nblk - 1)
    def _():
        out_ref[...] = out_ref[...] * (1.0 / N_FM)


def _fm_mean(x_fm, sq, cq, scp, ccp, srv, crv, p, wr_sum, bl_sum):
    blk, nblk = 400, 125
    grid = (nblk,)
    full = lambda *s: pl.BlockSpec(s, lambda i: (0,) * len(s))
    return pl.pallas_call(
        functools.partial(_fm_body, nblk=nblk),
        grid=grid,
        in_specs=[
            pl.BlockSpec((blk, H), lambda i: (i, 0)),
            pl.BlockSpec((2, 4, blk, 32), lambda i: (0, 0, i, 0)),
            pl.BlockSpec((2, blk, 16), lambda i: (0, i, 0)),
            pl.BlockSpec((2, 1, blk, 32), lambda i: (0, 0, i, 0)),
            pl.BlockSpec((2, blk, 16), lambda i: (0, i, 0)),
            pl.BlockSpec((2, 1, blk, 32), lambda i: (0, 0, i, 0)),
            pl.BlockSpec((2, blk, 16), lambda i: (0, i, 0)),
            full(H, H), full(H, 32), full(H, 32), full(H, H),
            full(1, H), full(1, H), full(1, H),
        ],
        out_specs=pl.BlockSpec((1, H), lambda i: (0, 0)),
        out_shape=jax.ShapeDtypeStruct((1, H), jnp.float32),
    )(x_fm, sq, cq, scp, ccp, srv, crv,
      p["Wl_qoq"], p["Wl_cp"], p["Wl_rev"], wr_sum, bl_sum,
      p["g_fm"].reshape(1, H), p["b_fm"].reshape(1, H))


def _tp_body(pe_ref, sb_ref, cb_ref, wlb_ref, wrb_ref, bl_ref, g_ref, b_ref,
             out_ref, nblk):
    i = pl.program_id(0)
    o = _chunk_matmul(sb_ref, _inv_counts(cb_ref), wlb_ref)
    o += jax.lax.dot_general(pe_ref[...], wrb_ref[...],
                             (((1,), (1,)), ((), ())),
                             preferred_element_type=jnp.float32)
    o += bl_ref[...]
    y = _ln_relu(o, g_ref[...], b_ref[...])
    s = jnp.sum(y, axis=0, keepdims=True)

    @pl.when(i == 0)
    def _():
        out_ref[...] = jnp.zeros_like(out_ref)
    out_ref[...] += s
    @pl.when(i == nblk - 1)
    def _():
        out_ref[...] = out_ref[...] * (1.0 / N_TP)


def _tp_mean(pe, sb, cb, p):
    blk, nblk = 512, 16
    full = lambda *s: pl.BlockSpec(s, lambda i: (0,) * len(s))
    return pl.pallas_call(
        functools.partial(_tp_body, nblk=nblk),
        grid=(nblk,),
        in_specs=[
            pl.BlockSpec((blk, 32), lambda i: (i, 0)),
            pl.BlockSpec((2, 4, blk, 32), lambda i: (0, 0, i, 0)),
            pl.BlockSpec((2, blk, 16), lambda i: (0, i, 0)),
            full(H, H), full(H, 32), full(1, H), full(1, H), full(1, H),
        ],
        out_specs=pl.BlockSpec((1, H), lambda i: (0, 0)),
        out_shape=jax.ShapeDtypeStruct((1, H), jnp.float32),
    )(pe, sb, cb, p["Wl_bel"], p["Wr_bel"], p["bl_bel"].reshape(1, H),
      p["g_tp"].reshape(1, H), p["b_tp"].reshape(1, H))


def _sm_body(x_ref, sd_ref, cd_ref, wld_ref, wrd_ref, bl_ref, g_ref, b_ref,
             out_ref, nblk):
    i = pl.program_id(0)
    o = _chunk_matmul(sd_ref, _inv_counts(cd_ref), wld_ref)
    o += jax.lax.dot_general(x_ref[...], wrd_ref[...],
                             (((1,), (1,)), ((), ())),
                             preferred_element_type=jnp.float32)
    o += bl_ref[...]
    y = _ln_relu(o, g_ref[...], b_ref[...])
    s = jnp.sum(y, axis=0, keepdims=True)

    @pl.when(i == 0)
    def _():
        out_ref[...] = jnp.zeros_like(out_ref)
    out_ref[...] += s
    @pl.when(i == nblk - 1)
    def _():
        out_ref[...] = out_ref[...] * (1.0 / N_SM)


def _sm_mean(x_sm, sd, cd, p):
    blk, nblk = 400, 50
    full = lambda *s: pl.BlockSpec(s, lambda i: (0,) * len(s))
    return pl.pallas_call(
        functools.partial(_sm_body, nblk=nblk),
        grid=(nblk,),
        in_specs=[
            pl.BlockSpec((blk, H), lambda i: (i, 0)),
            pl.BlockSpec((2, 1, blk, 32), lambda i: (0, 0, i, 0)),
            pl.BlockSpec((2, blk, 16), lambda i: (0, i, 0)),
            full(H, 32), full(H, H), full(1, H), full(1, H), full(1, H),
        ],
        out_specs=pl.BlockSpec((1, H), lambda i: (0, 0)),
        out_shape=jax.ShapeDtypeStruct((1, H), jnp.float32),
    )(x_sm, sd, cd, p["Wl_cd"], p["Wr_cd"], p["bl_cd"].reshape(1, H),
      p["g_sm"].reshape(1, H), p["b_sm"].reshape(1, H))


def _head_body(fm_ref, tp_ref, sm_ref, gf_ref, w1_ref, b1_ref, w2_ref,
               b2_ref, out_ref):
    h = jnp.concatenate(
        [fm_ref[...], tp_ref[...], sm_ref[...], gf_ref[...]], axis=1)
    h1 = jax.lax.dot_general(h, w1_ref[...], (((1,), (1,)), ((), ())),
                             preferred_element_type=jnp.float32)
    h1 = jnp.maximum(h1 + b1_ref[...], 0.0)
    out_ref[0, 0] = jnp.sum(h1 * w2_ref[...]) + b2_ref[0, 0]


def _head(fm, tp, sm, gf, p):
    return pl.pallas_call(
        _head_body,
        in_specs=[pl.BlockSpec(memory_space=pltpu.VMEM)] * 7
        + [pl.BlockSpec(memory_space=pltpu.SMEM)],
        out_specs=pl.BlockSpec(memory_space=pltpu.SMEM),
        out_shape=jax.ShapeDtypeStruct((1, 1), jnp.float32),
    )(fm, tp, sm, gf, p["W1"], p["b1"].reshape(1, 64), p["W2"],
      p["b2"].reshape(1, 1))


_SC_PARAMS = pltpu.CompilerParams(use_tc_tiling_on_sc=False)


@functools.lru_cache(maxsize=None)
def _sc_mesh():
    return plsc.VectorSubcoreMesh(core_axis_name="c", subcore_axis_name="s")

_CFG = {
    "qoq": (614400, P_FM, 4, 10),
    "bel": (401408, P_TP, 4, 14),
    "cp": (401408, P_FM, 1, 14),
    "cd": (204800, P_SM, 1, 10),
    "rev": (401408, P_FM, 1, 14),
}


def _zero_vmem(ref, rows, width):
    z = jnp.zeros((16,), jnp.float32)

    def zb(i, _):
        for w in range(width // 16):
            ref[i, pl.ds(w * 16, 16)] = z
        return _

    lax.fori_loop(0, rows, zb, None)


@functools.lru_cache(maxsize=None)
def _make_agg(e_pad, n_pad, k, g):
    ns = e_pad // 32 // 128
    nch = ns // g
    rpt = n_pad // 16
    zn = 16 if rpt % 16 == 0 else 8
    zr = rpt // zn

    nb = 4

    @functools.partial(
        pl.kernel, mesh=_sc_mesh(), compiler_params=_SC_PARAMS,
        out_type=jax.ShapeDtypeStruct((2, k, n_pad, 32), jnp.float32),
        scratch_types=[
            pltpu.VMEM((g, 128), jnp.int32),
            pltpu.VMEM((g, 128), jnp.int32),
            [pltpu.VMEM((128, 32), jnp.float32)] * nb,
            pltpu.VMEM((zr, 32), jnp.float32),
            pltpu.VMEM_SHARED((n_pad, 32), jnp.float32),
            [pltpu.SemaphoreType.DMA] * nb,
            [pltpu.SemaphoreType.DMA] * nb,
        ],
    )
    def agg(src3d, dst3d, *rest):
        tables = rest[:k]
        out = rest[k]
        src_c, dst_c, bufs, zv, acc, gsems, ssems = rest[k + 1:]
        sc = lax.axis_index("c")
        tl = lax.axis_index("s")
        wid = sc * 16 + tl
        base = tl * rpt

        _zero_vmem(zv, zr, 32)
        for z in range(zn):
            pltpu.sync_copy(zv, acc.at[pl.ds(base + z * zr, zr)])

        for c in range(k):
            plsc.subcore_barrier()
            tab = tables[c]

            def chunk(q, _):
                pltpu.sync_copy(src3d.at[wid, pl.ds(q * g, g)], src_c)
                pltpu.sync_copy(dst3d.at[wid, pl.ds(q * g, g)], dst_c)
                gd, sd = {}, {}
                for j in range(g):
                    b = j % nb
                    if j >= nb:
                        sd[j - nb].wait()
                    gd[j] = pltpu.async_copy(tab.at[src_c.at[j]], bufs[b],
                                             gsems[b])
                    if j >= 1:
                        jj = j - 1
                        gd[jj].wait()
                        sd[jj] = pltpu.async_copy(
                            bufs[jj % nb], acc.at[dst_c.at[jj]],
                            ssems[jj % nb], add=True)
                gd[g - 1].wait()
                sd[g - 1] = pltpu.async_copy(
                    bufs[(g - 1) % nb], acc.at[dst_c.at[g - 1]],
                    ssems[(g - 1) % nb], add=True)
                for jj in range(g - nb, g):
                    sd[jj].wait()
                return _

            lax.fori_loop(0, nch, chunk, None)
            plsc.subcore_barrier()
            pltpu.sync_copy(acc.at[pl.ds(base, rpt)],
                            out.at[sc, c, pl.ds(base, rpt)])
            if c + 1 < k:
                for z in range(zn):
                    pltpu.sync_copy(zv, acc.at[pl.ds(base + z * zr, zr)])

    return agg


@functools.lru_cache(maxsize=None)
def _make_counts(cfgs):
    max_np = max(c[1] for c in cfgs)
    max_g = max(c[2] for c in cfgs)
    rpt_max = max_np // 16
    zr = rpt_max // 8

    @functools.partial(
        pl.kernel, mesh=_sc_mesh(), compiler_params=_SC_PARAMS,
        out_type=tuple(jax.ShapeDtypeStruct((2, c[1], 16), jnp.float32)
                       for c in cfgs),
        scratch_types=[
            pltpu.VMEM((max_g, 128), jnp.int32),
            pltpu.VMEM((128, 16), jnp.float32),
            pltpu.VMEM((zr, 16), jnp.float32),
            pltpu.VMEM_SHARED((max_np, 16), jnp.float32),
            pltpu.SemaphoreType.DMA,
        ],
    )
    def counts(*args):
        n = len(cfgs)
        dsts = args[:n]
        outs = args[n:2 * n]
        dst_c, ones, zv, acc, sem = args[2 * n:]
        sc = lax.axis_index("c")
        tl = lax.axis_index("s")
        wid = sc * 16 + tl

        _zero_vmem(zv, zr, 16)
        one = jnp.full((16,), 1.0 / 16.0, jnp.float32)

        def ob(i, _):
            ones[i, pl.ds(0, 16)] = one
            return _

        lax.fori_loop(0, 128, ob, None)

        for r, (e_pad, np_, g) in enumerate(cfgs):
            ns = e_pad // 32 // 128
            nch = ns // g
            rpt = np_ // 16
            for z in range(8):
                pltpu.sync_copy(zv, acc.at[pl.ds(tl * rpt_max + z * zr, zr)])
            plsc.subcore_barrier()

            def chunk(q, _):
                pltpu.sync_copy(dsts[r].at[wid, pl.ds(q * g, g)],
                                dst_c.at[pl.ds(0, g)])
                sd = [pltpu.async_copy(ones, acc.at[dst_c.at[j]], sem,
                                       add=True) for j in range(g)]
                for d in sd:
                    d.wait()
                return _

            lax.fori_loop(0, nch, chunk, None)
            plsc.subcore_barrier()
            pltpu.sync_copy(acc.at[pl.ds(tl * rpt, rpt)],
                            outs[r].at[sc, pl.ds(tl * rpt, rpt)])
            if r + 1 < n:
                plsc.subcore_barrier()

    return counts


def _pad_edges(e, e_pad, n_dst):
    pad = e_pad - e.shape[1]
    src = jnp.concatenate([e[0], jnp.zeros((pad,), jnp.int32)])
    dst = jnp.concatenate([e[1], jnp.full((pad,), n_dst, jnp.int32)])
    return src.reshape(32, -1, 128), dst.reshape(32, -1, 128)


def kernel(x_fm, x_sm, gf, period_vol, edge_qoq, edge_bel, edge_cp, edge_cd,
           edge_rev, params):
    p = params
    pe, wr_sum, bl_sum = _prep(
        p["W_pe"], period_vol, p["Wr_qoq"], p["Wr_cp"], p["Wr_rev"],
        p["bl_qoq"].reshape(1, H), p["bl_cp"].reshape(1, H),
        p["bl_rev"].reshape(1, H))

    xc = tuple(x_fm[:, 32 * c:32 * (c + 1)] for c in range(4))
    edges = {"qoq": edge_qoq, "bel": edge_bel, "cp": edge_cp,
             "cd": edge_cd, "rev": edge_rev}
    ndst = {"qoq": N_FM, "bel": N_TP, "cp": N_FM, "cd": N_SM, "rev": N_FM}
    srcs, dsts = {}, {}
    for r, (e_pad, n_pad, k, g) in _CFG.items():
        srcs[r], dsts[r] = _pad_edges(edges[r], e_pad, ndst[r])

    rels = ["qoq", "bel", "cp", "cd", "rev"]
    cnt_cfg = tuple((_CFG[r][0], _CFG[r][1], _CFG[r][3]) for r in rels)
    cq, cb, ccp, cd, crv = _make_counts(cnt_cfg)(*[dsts[r] for r in rels])

    sq = _make_agg(*_CFG["qoq"])(srcs["qoq"], dsts["qoq"], *xc)
    sb = _make_agg(*_CFG["bel"])(srcs["bel"], dsts["bel"], *xc)
    scp = _make_agg(*_CFG["cp"])(srcs["cp"], dsts["cp"], pe)
    sd = _make_agg(*_CFG["cd"])(srcs["cd"], dsts["cd"], pe)
    srv = _make_agg(*_CFG["rev"])(srcs["rev"], dsts["rev"], pe)

    fm = _fm_mean(x_fm, sq, cq, scp, ccp, srv, crv, p, wr_sum, bl_sum)
    tp = _tp_mean(pe, sb, cb, p)
    sm = _sm_mean(x_sm, sd, cd, p)
    out = _head(fm, tp, sm, gf, p)
    return out.reshape(())


# speedup vs baseline: 6.2953x; 1.0695x over previous
import functools

import jax
import jax.numpy as jnp
from jax import lax
from jax.experimental import pallas as pl
from jax.experimental.pallas import tpu as pltpu
from jax.experimental.pallas import tpu_sc as plsc

N_FM = 50000
N_TP = 8192
N_SM = 20000
H = 128

P_FM = 50176
P_TP = 8320
P_SM = 20096


def _prep_body(wpe, pvol, wr_qoq, wr_cp, wr_rev, bl_qoq, bl_cp, bl_rev,
               pe_out, wr_out, bl_out):
    pe_out[...] = wpe[...] * pvol[...]
    wr_out[...] = wr_qoq[...] + wr_cp[...] + wr_rev[...]
    bl_out[...] = bl_qoq[...] + bl_cp[...] + bl_rev[...]


def _prep(wpe, pvol, wr_qoq, wr_cp, wr_rev, bl_qoq, bl_cp, bl_rev):
    return pl.pallas_call(
        _prep_body,
        out_shape=(
            jax.ShapeDtypeStruct((N_TP, 32), jnp.float32),
            jax.ShapeDtypeStruct((H, H), jnp.float32),
            jax.ShapeDtypeStruct((1, H), jnp.float32),
        ),
    )(wpe, pvol, wr_qoq, wr_cp, wr_rev, bl_qoq, bl_cp, bl_rev)


def _ln_relu(o, g, b):
    mu = jnp.mean(o, axis=1, keepdims=True)
    d = o - mu
    v = jnp.mean(d * d, axis=1, keepdims=True)
    y = d * lax.rsqrt(v + 1e-5) * g + b
    return jnp.maximum(y, 0.0)


def _inv_counts(cnt_ref):
    c = jnp.sum(cnt_ref[0] + cnt_ref[1], axis=1, keepdims=True)
    return 1.0 / jnp.maximum(c, 1.0)


def _chunk_matmul(s_ref, inv, wl_ref):
    k = s_ref.shape[1]
    out = None
    for c in range(k):
        agg = (s_ref[0, c] + s_ref[1, c]) * inv
        part = jax.lax.dot_general(
            agg, wl_ref[:, 32 * c:32 * c + 32],
            (((1,), (1,)), ((), ())),
            preferred_element_type=jnp.float32)
        out = part if out is None else out + part
    return out


def _fm_body(x_ref, sq_ref, cq_ref, sc_ref, cc_ref, sr_ref, cr_ref,
             wlq_ref, wlc_ref, wlr_ref, wr_ref, bl_ref, g_ref, b_ref,
             out_ref, nblk):
    i = pl.program_id(0)
    aggq = (sq_ref[0] + sq_ref[1]) * _inv_counts(cq_ref)
    o = jax.lax.dot_general(aggq, wlq_ref[...], (((1,), (1,)), ((), ())),
                            preferred_element_type=jnp.float32)
    o += _chunk_matmul(sc_ref, _inv_counts(cc_ref), wlc_ref)
    o += _chunk_matmul(sr_ref, _inv_counts(cr_ref), wlr_ref)
    o += jax.lax.dot_general(x_ref[...], wr_ref[...],
                             (((1,), (1,)), ((), ())),
                             preferred_element_type=jnp.float32)
    o = (o + bl_ref[...]) / 3.0
    y = _ln_relu(o, g_ref[...], b_ref[...])
    s = jnp.sum(y, axis=0, keepdims=True)

    @pl.when(i == 0)
    def _():
        out_ref[...] = jnp.zeros_like(out_ref)
    out_ref[...] += s
    @pl.when(i == nblk - 1)
    def _():
        out_ref[...] = out_ref[...] * (1.0 / N_FM)


def _fm_mean(x_fm, sq, cq, scp, ccp, srv, crv, p, wr_sum, bl_sum):
    blk, nblk = 400, 125
    grid = (nblk,)
    full = lambda *s: pl.BlockSpec(s, lambda i: (0,) * len(s))
    return pl.pallas_call(
        functools.partial(_fm_body, nblk=nblk),
        grid=grid,
        in_specs=[
            pl.BlockSpec((blk, H), lambda i: (i, 0)),
            pl.BlockSpec((2, blk, H), lambda i: (0, i, 0)),
            pl.BlockSpec((2, blk, 16), lambda i: (0, i, 0)),
            pl.BlockSpec((2, 1, blk, 32), lambda i: (0, 0, i, 0)),
            pl.BlockSpec((2, blk, 16), lambda i: (0, i, 0)),
            pl.BlockSpec((2, 1, blk, 32), lambda i: (0, 0, i, 0)),
            pl.BlockSpec((2, blk, 16), lambda i: (0, i, 0)),
            full(H, H), full(H, 32), full(H, 32), full(H, H),
            full(1, H), full(1, H), full(1, H),
        ],
        out_specs=pl.BlockSpec((1, H), lambda i: (0, 0)),
        out_shape=jax.ShapeDtypeStruct((1, H), jnp.float32),
    )(x_fm, sq, cq, scp, ccp, srv, crv,
      p["Wl_qoq"], p["Wl_cp"], p["Wl_rev"], wr_sum, bl_sum,
      p["g_fm"].reshape(1, H), p["b_fm"].reshape(1, H))


def _tp_body(pe_ref, sb_ref, cb_ref, wlb_ref, wrb_ref, bl_ref, g_ref, b_ref,
             out_ref, nblk):
    i = pl.program_id(0)
    aggb = (sb_ref[0] + sb_ref[1]) * _inv_counts(cb_ref)
    o = jax.lax.dot_general(aggb, wlb_ref[...], (((1,), (1,)), ((), ())),
                            preferred_element_type=jnp.float32)
    o += jax.lax.dot_general(pe_ref[...], wrb_ref[...],
                             (((1,), (1,)), ((), ())),
                             preferred_element_type=jnp.float32)
    o += bl_ref[...]
    y = _ln_relu(o, g_ref[...], b_ref[...])
    s = jnp.sum(y, axis=0, keepdims=True)

    @pl.when(i == 0)
    def _():
        out_ref[...] = jnp.zeros_like(out_ref)
    out_ref[...] += s
    @pl.when(i == nblk - 1)
    def _():
        out_ref[...] = out_ref[...] * (1.0 / N_TP)


def _tp_mean(pe, sb, cb, p):
    blk, nblk = 512, 16
    full = lambda *s: pl.BlockSpec(s, lambda i: (0,) * len(s))
    return pl.pallas_call(
        functools.partial(_tp_body, nblk=nblk),
        grid=(nblk,),
        in_specs=[
            pl.BlockSpec((blk, 32), lambda i: (i, 0)),
            pl.BlockSpec((2, blk, H), lambda i: (0, i, 0)),
            pl.BlockSpec((2, blk, 16), lambda i: (0, i, 0)),
            full(H, H), full(H, 32), full(1, H), full(1, H), full(1, H),
        ],
        out_specs=pl.BlockSpec((1, H), lambda i: (0, 0)),
        out_shape=jax.ShapeDtypeStruct((1, H), jnp.float32),
    )(pe, sb, cb, p["Wl_bel"], p["Wr_bel"], p["bl_bel"].reshape(1, H),
      p["g_tp"].reshape(1, H), p["b_tp"].reshape(1, H))


def _sm_body(x_ref, sd_ref, cd_ref, wld_ref, wrd_ref, bl_ref, g_ref, b_ref,
             out_ref, nblk):
    i = pl.program_id(0)
    o = _chunk_matmul(sd_ref, _inv_counts(cd_ref), wld_ref)
    o += jax.lax.dot_general(x_ref[...], wrd_ref[...],
                             (((1,), (1,)), ((), ())),
                             preferred_element_type=jnp.float32)
    o += bl_ref[...]
    y = _ln_relu(o, g_ref[...], b_ref[...])
    s = jnp.sum(y, axis=0, keepdims=True)

    @pl.when(i == 0)
    def _():
        out_ref[...] = jnp.zeros_like(out_ref)
    out_ref[...] += s
    @pl.when(i == nblk - 1)
    def _():
        out_ref[...] = out_ref[...] * (1.0 / N_SM)


def _sm_mean(x_sm, sd, cd, p):
    blk, nblk = 400, 50
    full = lambda *s: pl.BlockSpec(s, lambda i: (0,) * len(s))
    return pl.pallas_call(
        functools.partial(_sm_body, nblk=nblk),
        grid=(nblk,),
        in_specs=[
            pl.BlockSpec((blk, H), lambda i: (i, 0)),
            pl.BlockSpec((2, 1, blk, 32), lambda i: (0, 0, i, 0)),
            pl.BlockSpec((2, blk, 16), lambda i: (0, i, 0)),
            full(H, 32), full(H, H), full(1, H), full(1, H), full(1, H),
        ],
        out_specs=pl.BlockSpec((1, H), lambda i: (0, 0)),
        out_shape=jax.ShapeDtypeStruct((1, H), jnp.float32),
    )(x_sm, sd, cd, p["Wl_cd"], p["Wr_cd"], p["bl_cd"].reshape(1, H),
      p["g_sm"].reshape(1, H), p["b_sm"].reshape(1, H))


def _head_body(fm_ref, tp_ref, sm_ref, gf_ref, w1_ref, b1_ref, w2_ref,
               b2_ref, out_ref):
    h = jnp.concatenate(
        [fm_ref[...], tp_ref[...], sm_ref[...], gf_ref[...]], axis=1)
    h1 = jax.lax.dot_general(h, w1_ref[...], (((1,), (1,)), ((), ())),
                             preferred_element_type=jnp.float32)
    h1 = jnp.maximum(h1 + b1_ref[...], 0.0)
    out_ref[0, 0] = jnp.sum(h1 * w2_ref[...]) + b2_ref[0, 0]


def _head(fm, tp, sm, gf, p):
    return pl.pallas_call(
        _head_body,
        in_specs=[pl.BlockSpec(memory_space=pltpu.VMEM)] * 7
        + [pl.BlockSpec(memory_space=pltpu.SMEM)],
        out_specs=pl.BlockSpec(memory_space=pltpu.SMEM),
        out_shape=jax.ShapeDtypeStruct((1, 1), jnp.float32),
    )(fm, tp, sm, gf, p["W1"], p["b1"].reshape(1, 64), p["W2"],
      p["b2"].reshape(1, 1))


_SC_PARAMS = pltpu.CompilerParams(use_tc_tiling_on_sc=False)


@functools.lru_cache(maxsize=None)
def _sc_mesh():
    return plsc.VectorSubcoreMesh(core_axis_name="c", subcore_axis_name="s")

_CFG = {
    "qoq": (614400, P_FM, 4, 10),
    "bel": (401408, P_TP, 4, 14),
    "cp": (401408, P_FM, 1, 14),
    "cd": (204800, P_SM, 1, 10),
    "rev": (401408, P_FM, 1, 14),
}


def _zero_vmem(ref, rows, width):
    z = jnp.zeros((16,), jnp.float32)

    def zb(i, _):
        for w in range(width // 16):
            ref[i, pl.ds(w * 16, 16)] = z
        return _

    lax.fori_loop(0, rows, zb, None)


@functools.lru_cache(maxsize=None)
def _make_agg(e_pad, n_pad, k, g):
    ns = e_pad // 32 // 128
    nch = ns // g
    rpt = n_pad // 16
    zn = 16 if rpt % 16 == 0 else 8
    zr = rpt // zn

    nb = 4

    wide = (k == 4)
    oshape = (2, n_pad, 128) if wide else (2, k, n_pad, 32)

    @functools.partial(
        pl.kernel, mesh=_sc_mesh(), compiler_params=_SC_PARAMS,
        out_type=jax.ShapeDtypeStruct(oshape, jnp.float32),
        scratch_types=[
            pltpu.VMEM((g, 128), jnp.int32),
            pltpu.VMEM((g, 128), jnp.int32),
            [pltpu.VMEM((128, 32), jnp.float32)] * nb,
            pltpu.VMEM((zr, 32), jnp.float32),
            pltpu.VMEM_SHARED((n_pad, 32), jnp.float32),
            [pltpu.SemaphoreType.DMA] * nb,
            [pltpu.SemaphoreType.DMA] * nb,
        ],
    )
    def agg(src3d, dst3d, *rest):
        tables = rest[:k]
        out = rest[k]
        src_c, dst_c, bufs, zv, acc, gsems, ssems = rest[k + 1:]
        sc = lax.axis_index("c")
        tl = lax.axis_index("s")
        wid = sc * 16 + tl
        base = tl * rpt

        _zero_vmem(zv, zr, 32)
        for z in range(zn):
            pltpu.sync_copy(zv, acc.at[pl.ds(base + z * zr, zr)])

        for c in range(k):
            plsc.subcore_barrier()
            tab = tables[c]

            def chunk(q, _):
                pltpu.sync_copy(src3d.at[wid, pl.ds(q * g, g)], src_c)
                pltpu.sync_copy(dst3d.at[wid, pl.ds(q * g, g)], dst_c)
                gd, sd = {}, {}
                for j in range(g):
                    b = j % nb
                    if j >= nb:
                        sd[j - nb].wait()
                    gd[j] = pltpu.async_copy(tab.at[src_c.at[j]], bufs[b],
                                             gsems[b])
                    if j >= 1:
                        jj = j - 1
                        gd[jj].wait()
                        sd[jj] = pltpu.async_copy(
                            bufs[jj % nb], acc.at[dst_c.at[jj]],
                            ssems[jj % nb], add=True)
                gd[g - 1].wait()
                sd[g - 1] = pltpu.async_copy(
                    bufs[(g - 1) % nb], acc.at[dst_c.at[g - 1]],
                    ssems[(g - 1) % nb], add=True)
                for jj in range(g - nb, g):
                    sd[jj].wait()
                return _

            lax.fori_loop(0, nch, chunk, None)
            plsc.subcore_barrier()
            if wide:
                pltpu.sync_copy(acc.at[pl.ds(base, rpt)],
                                out.at[sc, pl.ds(base, rpt),
                                       pl.ds(32 * c, 32)])
            else:
                pltpu.sync_copy(acc.at[pl.ds(base, rpt)],
                                out.at[sc, c, pl.ds(base, rpt)])
            if c + 1 < k:
                for z in range(zn):
                    pltpu.sync_copy(zv, acc.at[pl.ds(base + z * zr, zr)])

    return agg


@functools.lru_cache(maxsize=None)
def _make_counts(cfgs):
    max_np = max(c[1] for c in cfgs)
    max_g = max(c[2] for c in cfgs)
    rpt_max = max_np // 16
    zr = rpt_max // 8

    @functools.partial(
        pl.kernel, mesh=_sc_mesh(), compiler_params=_SC_PARAMS,
        out_type=tuple(jax.ShapeDtypeStruct((2, c[1], 16), jnp.float32)
                       for c in cfgs),
        scratch_types=[
            pltpu.VMEM((max_g, 128), jnp.int32),
            pltpu.VMEM((128, 16), jnp.float32),
            pltpu.VMEM((zr, 16), jnp.float32),
            pltpu.VMEM_SHARED((max_np, 16), jnp.float32),
            pltpu.SemaphoreType.DMA,
        ],
    )
    def counts(*args):
        n = len(cfgs)
        dsts = args[:n]
        outs = args[n:2 * n]
        dst_c, ones, zv, acc, sem = args[2 * n:]
        sc = lax.axis_index("c")
        tl = lax.axis_index("s")
        wid = sc * 16 + tl

        _zero_vmem(zv, zr, 16)
        one = jnp.full((16,), 1.0 / 16.0, jnp.float32)

        def ob(i, _):
            ones[i, pl.ds(0, 16)] = one
            return _

        lax.fori_loop(0, 128, ob, None)

        for r, (e_pad, np_, g) in enumerate(cfgs):
            ns = e_pad // 32 // 128
            nch = ns // g
            rpt = np_ // 16
            for z in range(8):
                pltpu.sync_copy(zv, acc.at[pl.ds(tl * rpt_max + z * zr, zr)])
            plsc.subcore_barrier()

            def chunk(q, _):
                pltpu.sync_copy(dsts[r].at[wid, pl.ds(q * g, g)],
                                dst_c.at[pl.ds(0, g)])
                sd = [pltpu.async_copy(ones, acc.at[dst_c.at[j]], sem,
                                       add=True) for j in range(g)]
                for d in sd:
                    d.wait()
                return _

            lax.fori_loop(0, nch, chunk, None)
            plsc.subcore_barrier()
            pltpu.sync_copy(acc.at[pl.ds(tl * rpt, rpt)],
                            outs[r].at[sc, pl.ds(tl * rpt, rpt)])
            if r + 1 < n:
                plsc.subcore_barrier()

    return counts


def _pad_edges(e, e_pad, n_dst):
    pad = e_pad - e.shape[1]
    src = jnp.concatenate([e[0], jnp.zeros((pad,), jnp.int32)])
    dst = jnp.concatenate([e[1], jnp.full((pad,), n_dst, jnp.int32)])
    return src.reshape(32, -1, 128), dst.reshape(32, -1, 128)


def kernel(x_fm, x_sm, gf, period_vol, edge_qoq, edge_bel, edge_cp, edge_cd,
           edge_rev, params):
    p = params
    pe, wr_sum, bl_sum = _prep(
        p["W_pe"], period_vol, p["Wr_qoq"], p["Wr_cp"], p["Wr_rev"],
        p["bl_qoq"].reshape(1, H), p["bl_cp"].reshape(1, H),
        p["bl_rev"].reshape(1, H))

    xc = tuple(x_fm[:, 32 * c:32 * (c + 1)] for c in range(4))
    edges = {"qoq": edge_qoq, "bel": edge_bel, "cp": edge_cp,
             "cd": edge_cd, "rev": edge_rev}
    ndst = {"qoq": N_FM, "bel": N_TP, "cp": N_FM, "cd": N_SM, "rev": N_FM}
    srcs, dsts = {}, {}
    for r, (e_pad, n_pad, k, g) in _CFG.items():
        srcs[r], dsts[r] = _pad_edges(edges[r], e_pad, ndst[r])

    rels = ["qoq", "bel", "cp", "cd", "rev"]
    cnt_cfg = tuple((_CFG[r][0], _CFG[r][1], _CFG[r][3]) for r in rels)
    cq, cb, ccp, cd, crv = _make_counts(cnt_cfg)(*[dsts[r] for r in rels])

    sd = _make_agg(*_CFG["cd"])(srcs["cd"], dsts["cd"], pe)
    sb = _make_agg(*_CFG["bel"])(srcs["bel"], dsts["bel"], *xc)
    scp = _make_agg(*_CFG["cp"])(srcs["cp"], dsts["cp"], pe)
    srv = _make_agg(*_CFG["rev"])(srcs["rev"], dsts["rev"], pe)
    sq = _make_agg(*_CFG["qoq"])(srcs["qoq"], dsts["qoq"], *xc)

    fm = _fm_mean(x_fm, sq, cq, scp, ccp, srv, crv, p, wr_sum, bl_sum)
    tp = _tp_mean(pe, sb, cb, p)
    sm = _sm_mean(x_sm, sd, cd, p)
    out = _head(fm, tp, sm, gf, p)
    return out.reshape(())


# speedup vs baseline: 6.3061x; 1.0017x over previous
import functools

import jax
import jax.numpy as jnp
from jax import lax
from jax.experimental import pallas as pl
from jax.experimental.pallas import tpu as pltpu
from jax.experimental.pallas import tpu_sc as plsc

N_FM = 50000
N_TP = 8192
N_SM = 20000
H = 128

P_FM = 50176
P_TP = 8320
P_SM = 20096


def _prep_body(wpe, pvol, wr_qoq, wr_cp, wr_rev, bl_qoq, bl_cp, bl_rev,
               pe_out, wr_out, bl_out):
    pe_out[...] = wpe[...] * pvol[...]
    wr_out[...] = wr_qoq[...] + wr_cp[...] + wr_rev[...]
    bl_out[...] = bl_qoq[...] + bl_cp[...] + bl_rev[...]


def _prep(wpe, pvol, wr_qoq, wr_cp, wr_rev, bl_qoq, bl_cp, bl_rev):
    return pl.pallas_call(
        _prep_body,
        out_shape=(
            jax.ShapeDtypeStruct((N_TP, 32), jnp.float32),
            jax.ShapeDtypeStruct((H, H), jnp.float32),
            jax.ShapeDtypeStruct((1, H), jnp.float32),
        ),
    )(wpe, pvol, wr_qoq, wr_cp, wr_rev, bl_qoq, bl_cp, bl_rev)


def _ln_relu(o, g, b):
    mu = jnp.mean(o, axis=1, keepdims=True)
    d = o - mu
    v = jnp.mean(d * d, axis=1, keepdims=True)
    y = d * lax.rsqrt(v + 1e-5) * g + b
    return jnp.maximum(y, 0.0)


def _inv16(cnt, lo):
    c = jnp.sum(cnt[:, lo:lo + 16], axis=1, keepdims=True)
    return 1.0 / jnp.maximum(c, 1.0)


def _agg_matmul(sum2, inv, wl_ref):
    return jax.lax.dot_general(sum2 * inv, wl_ref[...],
                               (((1,), (1,)), ((), ())),
                               preferred_element_type=jnp.float32)


def _fm_body(x_ref, sq_ref, cfm_ref, sc_ref, sr_ref,
             wlq_ref, wlc_ref, wlr_ref, wr_ref, bl_ref, g_ref, b_ref,
             out_ref, nblk):
    i = pl.program_id(0)
    cnt = cfm_ref[0] + cfm_ref[1]
    o = _agg_matmul(sq_ref[0] + sq_ref[1], _inv16(cnt, 0), wlq_ref)
    o += _agg_matmul((sc_ref[0] + sc_ref[1])[:, 0:32], _inv16(cnt, 16),
                     wlc_ref)
    o += _agg_matmul((sr_ref[0] + sr_ref[1])[:, 0:32], _inv16(cnt, 32),
                     wlr_ref)
    o += jax.lax.dot_general(x_ref[...], wr_ref[...],
                             (((1,), (1,)), ((), ())),
                             preferred_element_type=jnp.float32)
    o = (o + bl_ref[...]) / 3.0
    y = _ln_relu(o, g_ref[...], b_ref[...])
    s = jnp.sum(y, axis=0, keepdims=True)

    @pl.when(i == 0)
    def _():
        out_ref[...] = jnp.zeros_like(out_ref)
    out_ref[...] += s
    @pl.when(i == nblk - 1)
    def _():
        out_ref[...] = out_ref[...] * (1.0 / N_FM)


def _fm_mean(x_fm, sq, cfm, scp, srv, p, wr_sum, bl_sum):
    blk, nblk = 400, 125
    grid = (nblk,)
    full = lambda *s: pl.BlockSpec(s, lambda i: (0,) * len(s))
    return pl.pallas_call(
        functools.partial(_fm_body, nblk=nblk),
        grid=grid,
        in_specs=[
            pl.BlockSpec((blk, H), lambda i: (i, 0)),
            pl.BlockSpec((2, blk, H), lambda i: (0, i, 0)),
            pl.BlockSpec((2, blk, H), lambda i: (0, i, 0)),
            pl.BlockSpec((2, blk, H), lambda i: (0, i, 0)),
            pl.BlockSpec((2, blk, H), lambda i: (0, i, 0)),
            full(H, H), full(H, 32), full(H, 32), full(H, H),
            full(1, H), full(1, H), full(1, H),
        ],
        out_specs=pl.BlockSpec((1, H), lambda i: (0, 0)),
        out_shape=jax.ShapeDtypeStruct((1, H), jnp.float32),
    )(x_fm, sq, cfm, scp, srv,
      p["Wl_qoq"], p["Wl_cp"], p["Wl_rev"], wr_sum, bl_sum,
      p["g_fm"].reshape(1, H), p["b_fm"].reshape(1, H))


def _tp_body(pe_ref, sb_ref, cb_ref, wlb_ref, wrb_ref, bl_ref, g_ref, b_ref,
             out_ref, nblk):
    i = pl.program_id(0)
    o = _agg_matmul(sb_ref[0] + sb_ref[1], _inv16(cb_ref[0] + cb_ref[1], 0),
                    wlb_ref)
    o += jax.lax.dot_general(pe_ref[...], wrb_ref[...],
                             (((1,), (1,)), ((), ())),
                             preferred_element_type=jnp.float32)
    o += bl_ref[...]
    y = _ln_relu(o, g_ref[...], b_ref[...])
    s = jnp.sum(y, axis=0, keepdims=True)

    @pl.when(i == 0)
    def _():
        out_ref[...] = jnp.zeros_like(out_ref)
    out_ref[...] += s
    @pl.when(i == nblk - 1)
    def _():
        out_ref[...] = out_ref[...] * (1.0 / N_TP)


def _tp_mean(pe, sb, cb, p):
    blk, nblk = 512, 16
    full = lambda *s: pl.BlockSpec(s, lambda i: (0,) * len(s))
    return pl.pallas_call(
        functools.partial(_tp_body, nblk=nblk),
        grid=(nblk,),
        in_specs=[
            pl.BlockSpec((blk, 32), lambda i: (i, 0)),
            pl.BlockSpec((2, blk, H), lambda i: (0, i, 0)),
            pl.BlockSpec((2, blk, H), lambda i: (0, i, 0)),
            full(H, H), full(H, 32), full(1, H), full(1, H), full(1, H),
        ],
        out_specs=pl.BlockSpec((1, H), lambda i: (0, 0)),
        out_shape=jax.ShapeDtypeStruct((1, H), jnp.float32),
    )(pe, sb, cb, p["Wl_bel"], p["Wr_bel"], p["bl_bel"].reshape(1, H),
      p["g_tp"].reshape(1, H), p["b_tp"].reshape(1, H))


def _sm_body(x_ref, sd_ref, cd_ref, wld_ref, wrd_ref, bl_ref, g_ref, b_ref,
             out_ref, nblk):
    i = pl.program_id(0)
    o = _agg_matmul((sd_ref[0] + sd_ref[1])[:, 0:32],
                    _inv16(cd_ref[0] + cd_ref[1], 0), wld_ref)
    o += jax.lax.dot_general(x_ref[...], wrd_ref[...],
                             (((1,), (1,)), ((), ())),
                             preferred_element_type=jnp.float32)
    o += bl_ref[...]
    y = _ln_relu(o, g_ref[...], b_ref[...])
    s = jnp.sum(y, axis=0, keepdims=True)

    @pl.when(i == 0)
    def _():
        out_ref[...] = jnp.zeros_like(out_ref)
    out_ref[...] += s
    @pl.when(i == nblk - 1)
    def _():
        out_ref[...] = out_ref[...] * (1.0 / N_SM)


def _sm_mean(x_sm, sd, cd, p):
    blk, nblk = 400, 50
    full = lambda *s: pl.BlockSpec(s, lambda i: (0,) * len(s))
    return pl.pallas_call(
        functools.partial(_sm_body, nblk=nblk),
        grid=(nblk,),
        in_specs=[
            pl.BlockSpec((blk, H), lambda i: (i, 0)),
            pl.BlockSpec((2, blk, H), lambda i: (0, i, 0)),
            pl.BlockSpec((2, blk, H), lambda i: (0, i, 0)),
            full(H, 32), full(H, H), full(1, H), full(1, H), full(1, H),
        ],
        out_specs=pl.BlockSpec((1, H), lambda i: (0, 0)),
        out_shape=jax.ShapeDtypeStruct((1, H), jnp.float32),
    )(x_sm, sd, cd, p["Wl_cd"], p["Wr_cd"], p["bl_cd"].reshape(1, H),
      p["g_sm"].reshape(1, H), p["b_sm"].reshape(1, H))


def _head_body(fm_ref, tp_ref, sm_ref, gf_ref, w1_ref, b1_ref, w2_ref,
               b2_ref, out_ref):
    h = jnp.concatenate(
        [fm_ref[...], tp_ref[...], sm_ref[...], gf_ref[...]], axis=1)
    h1 = jax.lax.dot_general(h, w1_ref[...], (((1,), (1,)), ((), ())),
                             preferred_element_type=jnp.float32)
    h1 = jnp.maximum(h1 + b1_ref[...], 0.0)
    out_ref[0, 0] = jnp.sum(h1 * w2_ref[...]) + b2_ref[0, 0]


def _head(fm, tp, sm, gf, p):
    return pl.pallas_call(
        _head_body,
        in_specs=[pl.BlockSpec(memory_space=pltpu.VMEM)] * 7
        + [pl.BlockSpec(memory_space=pltpu.SMEM)],
        out_specs=pl.BlockSpec(memory_space=pltpu.SMEM),
        out_shape=jax.ShapeDtypeStruct((1, 1), jnp.float32),
    )(fm, tp, sm, gf, p["W1"], p["b1"].reshape(1, 64), p["W2"],
      p["b2"].reshape(1, 1))


_SC_PARAMS = pltpu.CompilerParams(use_tc_tiling_on_sc=False)


@functools.lru_cache(maxsize=None)
def _sc_mesh():
    return plsc.VectorSubcoreMesh(core_axis_name="c", subcore_axis_name="s")

_CFG = {
    "qoq": (614400, P_FM, 4, 10),
    "bel": (401408, P_TP, 4, 14),
    "cp": (401408, P_FM, 1, 14),
    "cd": (204800, P_SM, 1, 10),
    "rev": (401408, P_FM, 1, 14),
}


def _zero_vmem(ref, rows, width):
    z = jnp.zeros((16,), jnp.float32)

    def zb(i, _):
        for w in range(width // 16):
            ref[i, pl.ds(w * 16, 16)] = z
        return _

    lax.fori_loop(0, rows, zb, None)


@functools.lru_cache(maxsize=None)
def _make_agg(e_pad, n_pad, k, g):
    ns = e_pad // 32 // 128
    nch = ns // g
    rpt = n_pad // 16
    zn = 16 if rpt % 16 == 0 else 8
    zr = rpt // zn

    nb = 4

    @functools.partial(
        pl.kernel, mesh=_sc_mesh(), compiler_params=_SC_PARAMS,
        out_type=jax.ShapeDtypeStruct((2, n_pad, 128), jnp.float32),
        scratch_types=[
            pltpu.VMEM((g, 128), jnp.int32),
            pltpu.VMEM((g, 128), jnp.int32),
            [pltpu.VMEM((128, 32), jnp.float32)] * nb,
            pltpu.VMEM((zr, 32), jnp.float32),
            pltpu.VMEM_SHARED((n_pad, 32), jnp.float32),
            [pltpu.SemaphoreType.DMA] * nb,
            [pltpu.SemaphoreType.DMA] * nb,
        ],
    )
    def agg(src3d, dst3d, *rest):
        tables = rest[:k]
        out = rest[k]
        src_c, dst_c, bufs, zv, acc, gsems, ssems = rest[k + 1:]
        sc = lax.axis_index("c")
        tl = lax.axis_index("s")
        wid = sc * 16 + tl
        base = tl * rpt

        _zero_vmem(zv, zr, 32)
        for z in range(zn):
            pltpu.sync_copy(zv, acc.at[pl.ds(base + z * zr, zr)])

        for c in range(k):
            plsc.subcore_barrier()
            tab = tables[c]

            def chunk(q, _):
                pltpu.sync_copy(src3d.at[wid, pl.ds(q * g, g)], src_c)
                pltpu.sync_copy(dst3d.at[wid, pl.ds(q * g, g)], dst_c)
                gd, sd = {}, {}
                for j in range(g):
                    b = j % nb
                    if j >= nb:
                        sd[j - nb].wait()
                    gd[j] = pltpu.async_copy(tab.at[src_c.at[j]], bufs[b],
                                             gsems[b])
                    if j >= 1:
                        jj = j - 1
                        gd[jj].wait()
                        sd[jj] = pltpu.async_copy(
                            bufs[jj % nb], acc.at[dst_c.at[jj]],
                            ssems[jj % nb], add=True)
                gd[g - 1].wait()
                sd[g - 1] = pltpu.async_copy(
                    bufs[(g - 1) % nb], acc.at[dst_c.at[g - 1]],
                    ssems[(g - 1) % nb], add=True)
                for jj in range(g - nb, g):
                    sd[jj].wait()
                return _

            lax.fori_loop(0, nch, chunk, None)
            plsc.subcore_barrier()
            pltpu.sync_copy(acc.at[pl.ds(base, rpt)],
                            out.at[sc, pl.ds(base, rpt),
                                   pl.ds(32 * c, 32)])
            if c + 1 < k:
                for z in range(zn):
                    pltpu.sync_copy(zv, acc.at[pl.ds(base + z * zr, zr)])

    return agg


@functools.lru_cache(maxsize=None)
def _make_counts(cfgs):
    out_npads = {}
    for _, np_, _, oi, _ in cfgs:
        out_npads[oi] = np_
    n_out = len(out_npads)
    max_np = max(c[1] for c in cfgs)
    max_g = max(c[2] for c in cfgs)
    rpt_max = max_np // 16
    zr = rpt_max // 8

    @functools.partial(
        pl.kernel, mesh=_sc_mesh(), compiler_params=_SC_PARAMS,
        out_type=tuple(jax.ShapeDtypeStruct((2, out_npads[i], 128),
                                            jnp.float32)
                       for i in range(n_out)),
        scratch_types=[
            pltpu.VMEM((max_g, 128), jnp.int32),
            pltpu.VMEM((128, 16), jnp.float32),
            pltpu.VMEM((zr, 16), jnp.float32),
            pltpu.VMEM_SHARED((max_np, 16), jnp.float32),
            pltpu.SemaphoreType.DMA,
        ],
    )
    def counts(*args):
        n = len(cfgs)
        dsts = args[:n]
        outs = args[n:n + n_out]
        dst_c, ones, zv, acc, sem = args[n + n_out:]
        sc = lax.axis_index("c")
        tl = lax.axis_index("s")
        wid = sc * 16 + tl

        _zero_vmem(zv, zr, 16)
        one = jnp.full((16,), 1.0 / 16.0, jnp.float32)

        def ob(i, _):
            ones[i, pl.ds(0, 16)] = one
            return _

        lax.fori_loop(0, 128, ob, None)

        for r, (e_pad, np_, g, oi, col) in enumerate(cfgs):
            ns = e_pad // 32 // 128
            nch = ns // g
            rpt = np_ // 16
            for z in range(8):
                pltpu.sync_copy(zv, acc.at[pl.ds(tl * rpt_max + z * zr, zr)])
            plsc.subcore_barrier()

            def chunk(q, _):
                pltpu.sync_copy(dsts[r].at[wid, pl.ds(q * g, g)],
                                dst_c.at[pl.ds(0, g)])
                sd = [pltpu.async_copy(ones, acc.at[dst_c.at[j]], sem,
                                       add=True) for j in range(g)]
                for d in sd:
                    d.wait()
                return _

            lax.fori_loop(0, nch, chunk, None)
            plsc.subcore_barrier()
            pltpu.sync_copy(acc.at[pl.ds(tl * rpt, rpt)],
                            outs[oi].at[sc, pl.ds(tl * rpt, rpt),
                                        pl.ds(16 * col, 16)])
            if r + 1 < n:
                plsc.subcore_barrier()

    return counts


def _pad_edges(e, e_pad, n_dst):
    pad = e_pad - e.shape[1]
    src = jnp.concatenate([e[0], jnp.zeros((pad,), jnp.int32)])
    dst = jnp.concatenate([e[1], jnp.full((pad,), n_dst, jnp.int32)])
    return src.reshape(32, -1, 128), dst.reshape(32, -1, 128)


def kernel(x_fm, x_sm, gf, period_vol, edge_qoq, edge_bel, edge_cp, edge_cd,
           edge_rev, params):
    p = params
    pe, wr_sum, bl_sum = _prep(
        p["W_pe"], period_vol, p["Wr_qoq"], p["Wr_cp"], p["Wr_rev"],
        p["bl_qoq"].reshape(1, H), p["bl_cp"].reshape(1, H),
        p["bl_rev"].reshape(1, H))

    xc = tuple(x_fm[:, 32 * c:32 * (c + 1)] for c in range(4))
    edges = {"qoq": edge_qoq, "bel": edge_bel, "cp": edge_cp,
             "cd": edge_cd, "rev": edge_rev}
    ndst = {"qoq": N_FM, "bel": N_TP, "cp": N_FM, "cd": N_SM, "rev": N_FM}
    srcs, dsts = {}, {}
    for r, (e_pad, n_pad, k, g) in _CFG.items():
        srcs[r], dsts[r] = _pad_edges(edges[r], e_pad, ndst[r])

    rels = ["qoq", "bel", "cp", "cd", "rev"]
    slot = {"qoq": (0, 0), "bel": (1, 0), "cp": (0, 1), "cd": (2, 0),
            "rev": (0, 2)}
    cnt_cfg = tuple((_CFG[r][0], _CFG[r][1], _CFG[r][3]) + slot[r]
                    for r in rels)
    cfm, ctp, csm = _make_counts(cnt_cfg)(*[dsts[r] for r in rels])

    sd = _make_agg(*_CFG["cd"])(srcs["cd"], dsts["cd"], pe)
    sb = _make_agg(*_CFG["bel"])(srcs["bel"], dsts["bel"], *xc)
    scp = _make_agg(*_CFG["cp"])(srcs["cp"], dsts["cp"], pe)
    srv = _make_agg(*_CFG["rev"])(srcs["rev"], dsts["rev"], pe)
    sq = _make_agg(*_CFG["qoq"])(srcs["qoq"], dsts["qoq"], *xc)

    fm = _fm_mean(x_fm, sq, cfm, scp, srv, p, wr_sum, bl_sum)
    tp = _tp_mean(pe, sb, ctp, p)
    sm = _sm_mean(x_sm, sd, csm, p)
    out = _head(fm, tp, sm, gf, p)
    return out.reshape(())


# speedup vs baseline: 8.1108x; 1.2862x over previous
import functools

import jax
import jax.numpy as jnp
from jax import lax
from jax.experimental import pallas as pl
from jax.experimental.pallas import tpu as pltpu
from jax.experimental.pallas import tpu_sc as plsc

N_FM = 50000
N_TP = 8192
N_SM = 20000
H = 128

P_FM = 50176
P_TP = 8320
P_SM = 20096


def _prep_body(wpe, pvol, wr_qoq, wr_cp, wr_rev, bl_qoq, bl_cp, bl_rev,
               pe_out, wr_out, bl_out):
    pe_out[...] = wpe[...] * pvol[...]
    wr_out[...] = wr_qoq[...] + wr_cp[...] + wr_rev[...]
    bl_out[...] = bl_qoq[...] + bl_cp[...] + bl_rev[...]


def _prep(wpe, pvol, wr_qoq, wr_cp, wr_rev, bl_qoq, bl_cp, bl_rev):
    return pl.pallas_call(
        _prep_body,
        out_shape=(
            jax.ShapeDtypeStruct((N_TP, 32), jnp.float32),
            jax.ShapeDtypeStruct((H, H), jnp.float32),
            jax.ShapeDtypeStruct((1, H), jnp.float32),
        ),
    )(wpe, pvol, wr_qoq, wr_cp, wr_rev, bl_qoq, bl_cp, bl_rev)


def _ln_relu(o, g, b):
    mu = jnp.mean(o, axis=1, keepdims=True)
    d = o - mu
    v = jnp.mean(d * d, axis=1, keepdims=True)
    y = d * lax.rsqrt(v + 1e-5) * g + b
    return jnp.maximum(y, 0.0)


def _inv16(cnt, lo):
    c = jnp.sum(cnt[:, lo:lo + 16], axis=1, keepdims=True)
    return 1.0 / jnp.maximum(c, 1.0)


def _agg_matmul(sum2, inv, wl_ref):
    return jax.lax.dot_general(sum2 * inv, wl_ref[...],
                               (((1,), (1,)), ((), ())),
                               preferred_element_type=jnp.float32)


def _fm_body(x_ref, sq_ref, cfm_ref, sc_ref, sr_ref,
             wlq_ref, wlc_ref, wlr_ref, wr_ref, bl_ref, g_ref, b_ref,
             out_ref, nblk):
    i = pl.program_id(0)
    cnt = cfm_ref[0] + cfm_ref[1]
    o = _agg_matmul(sq_ref[0].astype(jnp.float32)
                    + sq_ref[1].astype(jnp.float32), _inv16(cnt, 0), wlq_ref)
    o += _agg_matmul((sc_ref[0].astype(jnp.float32)
                      + sc_ref[1].astype(jnp.float32))[:, 0:32],
                     _inv16(cnt, 16), wlc_ref)
    o += _agg_matmul((sr_ref[0].astype(jnp.float32)
                      + sr_ref[1].astype(jnp.float32))[:, 0:32],
                     _inv16(cnt, 32), wlr_ref)
    o += jax.lax.dot_general(x_ref[...], wr_ref[...],
                             (((1,), (1,)), ((), ())),
                             preferred_element_type=jnp.float32)
    o = (o + bl_ref[...]) / 3.0
    y = _ln_relu(o, g_ref[...], b_ref[...])
    s = jnp.sum(y, axis=0, keepdims=True)

    @pl.when(i == 0)
    def _():
        out_ref[...] = jnp.zeros_like(out_ref)
    out_ref[...] += s
    @pl.when(i == nblk - 1)
    def _():
        out_ref[...] = out_ref[...] * (1.0 / N_FM)


def _fm_mean(x_fm, sq, cfm, scp, srv, p, wr_sum, bl_sum):
    blk, nblk = 400, 125
    grid = (nblk,)
    full = lambda *s: pl.BlockSpec(s, lambda i: (0,) * len(s))
    return pl.pallas_call(
        functools.partial(_fm_body, nblk=nblk),
        grid=grid,
        in_specs=[
            pl.BlockSpec((blk, H), lambda i: (i, 0)),
            pl.BlockSpec((2, blk, H), lambda i: (0, i, 0)),
            pl.BlockSpec((2, blk, H), lambda i: (0, i, 0)),
            pl.BlockSpec((2, blk, H), lambda i: (0, i, 0)),
            pl.BlockSpec((2, blk, H), lambda i: (0, i, 0)),
            full(H, H), full(H, 32), full(H, 32), full(H, H),
            full(1, H), full(1, H), full(1, H),
        ],
        out_specs=pl.BlockSpec((1, H), lambda i: (0, 0)),
        out_shape=jax.ShapeDtypeStruct((1, H), jnp.float32),
    )(x_fm, sq, cfm, scp, srv,
      p["Wl_qoq"], p["Wl_cp"], p["Wl_rev"], wr_sum, bl_sum,
      p["g_fm"].reshape(1, H), p["b_fm"].reshape(1, H))


def _tp_body(pe_ref, sb_ref, cb_ref, wlb_ref, wrb_ref, bl_ref, g_ref, b_ref,
             out_ref, nblk):
    i = pl.program_id(0)
    o = _agg_matmul(sb_ref[0].astype(jnp.float32)
                    + sb_ref[1].astype(jnp.float32),
                    _inv16(cb_ref[0] + cb_ref[1], 0), wlb_ref)
    o += jax.lax.dot_general(pe_ref[...], wrb_ref[...],
                             (((1,), (1,)), ((), ())),
                             preferred_element_type=jnp.float32)
    o += bl_ref[...]
    y = _ln_relu(o, g_ref[...], b_ref[...])
    s = jnp.sum(y, axis=0, keepdims=True)

    @pl.when(i == 0)
    def _():
        out_ref[...] = jnp.zeros_like(out_ref)
    out_ref[...] += s
    @pl.when(i == nblk - 1)
    def _():
        out_ref[...] = out_ref[...] * (1.0 / N_TP)


def _tp_mean(pe, sb, cb, p):
    blk, nblk = 512, 16
    full = lambda *s: pl.BlockSpec(s, lambda i: (0,) * len(s))
    return pl.pallas_call(
        functools.partial(_tp_body, nblk=nblk),
        grid=(nblk,),
        in_specs=[
            pl.BlockSpec((blk, 32), lambda i: (i, 0)),
            pl.BlockSpec((2, blk, H), lambda i: (0, i, 0)),
            pl.BlockSpec((2, blk, H), lambda i: (0, i, 0)),
            full(H, H), full(H, 32), full(1, H), full(1, H), full(1, H),
        ],
        out_specs=pl.BlockSpec((1, H), lambda i: (0, 0)),
        out_shape=jax.ShapeDtypeStruct((1, H), jnp.float32),
    )(pe, sb, cb, p["Wl_bel"], p["Wr_bel"], p["bl_bel"].reshape(1, H),
      p["g_tp"].reshape(1, H), p["b_tp"].reshape(1, H))


def _sm_body(x_ref, sd_ref, cd_ref, wld_ref, wrd_ref, bl_ref, g_ref, b_ref,
             out_ref, nblk):
    i = pl.program_id(0)
    o = _agg_matmul((sd_ref[0].astype(jnp.float32)
                     + sd_ref[1].astype(jnp.float32))[:, 0:32],
                    _inv16(cd_ref[0] + cd_ref[1], 0), wld_ref)
    o += jax.lax.dot_general(x_ref[...], wrd_ref[...],
                             (((1,), (1,)), ((), ())),
                             preferred_element_type=jnp.float32)
    o += bl_ref[...]
    y = _ln_relu(o, g_ref[...], b_ref[...])
    s = jnp.sum(y, axis=0, keepdims=True)

    @pl.when(i == 0)
    def _():
        out_ref[...] = jnp.zeros_like(out_ref)
    out_ref[...] += s
    @pl.when(i == nblk - 1)
    def _():
        out_ref[...] = out_ref[...] * (1.0 / N_SM)


def _sm_mean(x_sm, sd, cd, p):
    blk, nblk = 400, 50
    full = lambda *s: pl.BlockSpec(s, lambda i: (0,) * len(s))
    return pl.pallas_call(
        functools.partial(_sm_body, nblk=nblk),
        grid=(nblk,),
        in_specs=[
            pl.BlockSpec((blk, H), lambda i: (i, 0)),
            pl.BlockSpec((2, blk, H), lambda i: (0, i, 0)),
            pl.BlockSpec((2, blk, H), lambda i: (0, i, 0)),
            full(H, 32), full(H, H), full(1, H), full(1, H), full(1, H),
        ],
        out_specs=pl.BlockSpec((1, H), lambda i: (0, 0)),
        out_shape=jax.ShapeDtypeStruct((1, H), jnp.float32),
    )(x_sm, sd, cd, p["Wl_cd"], p["Wr_cd"], p["bl_cd"].reshape(1, H),
      p["g_sm"].reshape(1, H), p["b_sm"].reshape(1, H))


def _head_body(fm_ref, tp_ref, sm_ref, gf_ref, w1_ref, b1_ref, w2_ref,
               b2_ref, out_ref):
    h = jnp.concatenate(
        [fm_ref[...], tp_ref[...], sm_ref[...], gf_ref[...]], axis=1)
    h1 = jax.lax.dot_general(h, w1_ref[...], (((1,), (1,)), ((), ())),
                             preferred_element_type=jnp.float32)
    h1 = jnp.maximum(h1 + b1_ref[...], 0.0)
    out_ref[0, 0] = jnp.sum(h1 * w2_ref[...]) + b2_ref[0, 0]


def _head(fm, tp, sm, gf, p):
    return pl.pallas_call(
        _head_body,
        in_specs=[pl.BlockSpec(memory_space=pltpu.VMEM)] * 7
        + [pl.BlockSpec(memory_space=pltpu.SMEM)],
        out_specs=pl.BlockSpec(memory_space=pltpu.SMEM),
        out_shape=jax.ShapeDtypeStruct((1, 1), jnp.float32),
    )(fm, tp, sm, gf, p["W1"], p["b1"].reshape(1, 64), p["W2"],
      p["b2"].reshape(1, 1))


_SC_PARAMS = pltpu.CompilerParams(use_tc_tiling_on_sc=False)


@functools.lru_cache(maxsize=None)
def _sc_mesh():
    return plsc.VectorSubcoreMesh(core_axis_name="c", subcore_axis_name="s")

_CFG = {
    "qoq": (614400, P_FM, 4, 10),
    "bel": (401408, P_TP, 4, 14),
    "cp": (401408, P_FM, 1, 14),
    "cd": (204800, P_SM, 1, 10),
    "rev": (401408, P_FM, 1, 14),
}


def _zero_vmem(ref, rows, width):
    z = jnp.zeros((16,), jnp.float32)

    def zb(i, _):
        for w in range(width // 16):
            ref[i, pl.ds(w * 16, 16)] = z
        return _

    lax.fori_loop(0, rows, zb, None)


@functools.lru_cache(maxsize=None)
def _make_agg(e_pad, n_pad, k, g):
    ns = e_pad // 32 // 128
    nch = ns // g
    rpt = n_pad // 16
    zn = 16 if rpt % 16 == 0 else 8
    zr = rpt // zn

    nb = 4

    @functools.partial(
        pl.kernel, mesh=_sc_mesh(), compiler_params=_SC_PARAMS,
        out_type=jax.ShapeDtypeStruct((2, n_pad, 128), jnp.bfloat16),
        scratch_types=[
            pltpu.VMEM((g, 128), jnp.int32),
            pltpu.VMEM((g, 128), jnp.int32),
            [pltpu.VMEM((128, 32), jnp.bfloat16)] * nb,
            pltpu.VMEM((zr, 32), jnp.bfloat16),
            pltpu.VMEM_SHARED((n_pad, 32), jnp.bfloat16),
            [pltpu.SemaphoreType.DMA] * nb,
            [pltpu.SemaphoreType.DMA] * nb,
        ],
    )
    def agg(src3d, dst3d, *rest):
        tables = rest[:k]
        out = rest[k]
        src_c, dst_c, bufs, zv, acc, gsems, ssems = rest[k + 1:]
        sc = lax.axis_index("c")
        tl = lax.axis_index("s")
        wid = sc * 16 + tl
        base = tl * rpt

        zb16 = jnp.zeros((32,), jnp.bfloat16)

        def _zb(i, _):
            zv[i, pl.ds(0, 32)] = zb16
            return _

        lax.fori_loop(0, zr, _zb, None)
        for z in range(zn):
            pltpu.sync_copy(zv, acc.at[pl.ds(base + z * zr, zr)])

        for c in range(k):
            plsc.subcore_barrier()
            tab = tables[c]

            def chunk(q, _):
                pltpu.sync_copy(src3d.at[wid, pl.ds(q * g, g)], src_c)
                pltpu.sync_copy(dst3d.at[wid, pl.ds(q * g, g)], dst_c)
                gd, sd = {}, {}
                for j in range(g):
                    b = j % nb
                    if j >= nb:
                        sd[j - nb].wait()
                    gd[j] = pltpu.async_copy(tab.at[src_c.at[j]], bufs[b],
                                             gsems[b])
                    if j >= 1:
                        jj = j - 1
                        gd[jj].wait()
                        sd[jj] = pltpu.async_copy(
                            bufs[jj % nb], acc.at[dst_c.at[jj]],
                            ssems[jj % nb], add=True)
                gd[g - 1].wait()
                sd[g - 1] = pltpu.async_copy(
                    bufs[(g - 1) % nb], acc.at[dst_c.at[g - 1]],
                    ssems[(g - 1) % nb], add=True)
                for jj in range(g - nb, g):
                    sd[jj].wait()
                return _

            lax.fori_loop(0, nch, chunk, None)
            plsc.subcore_barrier()
            pltpu.sync_copy(acc.at[pl.ds(base, rpt)],
                            out.at[sc, pl.ds(base, rpt),
                                   pl.ds(32 * c, 32)])
            if c + 1 < k:
                for z in range(zn):
                    pltpu.sync_copy(zv, acc.at[pl.ds(base + z * zr, zr)])

    return agg


@functools.lru_cache(maxsize=None)
def _make_counts(cfgs):
    out_npads = {}
    for _, np_, _, oi, _ in cfgs:
        out_npads[oi] = np_
    n_out = len(out_npads)
    max_np = max(c[1] for c in cfgs)
    max_g = max(c[2] for c in cfgs)
    rpt_max = max_np // 16
    zr = rpt_max // 8

    @functools.partial(
        pl.kernel, mesh=_sc_mesh(), compiler_params=_SC_PARAMS,
        out_type=tuple(jax.ShapeDtypeStruct((2, out_npads[i], 128),
                                            jnp.float32)
                       for i in range(n_out)),
        scratch_types=[
            pltpu.VMEM((max_g, 128), jnp.int32),
            pltpu.VMEM((128, 16), jnp.float32),
            pltpu.VMEM((zr, 16), jnp.float32),
            pltpu.VMEM_SHARED((max_np, 16), jnp.float32),
            pltpu.SemaphoreType.DMA,
        ],
    )
    def counts(*args):
        n = len(cfgs)
        dsts = args[:n]
        outs = args[n:n + n_out]
        dst_c, ones, zv, acc, sem = args[n + n_out:]
        sc = lax.axis_index("c")
        tl = lax.axis_index("s")
        wid = sc * 16 + tl

        _zero_vmem(zv, zr, 16)
        one = jnp.full((16,), 1.0 / 16.0, jnp.float32)

        def ob(i, _):
            ones[i, pl.ds(0, 16)] = one
            return _

        lax.fori_loop(0, 128, ob, None)

        for r, (e_pad, np_, g, oi, col) in enumerate(cfgs):
            ns = e_pad // 32 // 128
            nch = ns // g
            rpt = np_ // 16
            for z in range(8):
                pltpu.sync_copy(zv, acc.at[pl.ds(tl * rpt_max + z * zr, zr)])
            plsc.subcore_barrier()

            def chunk(q, _):
                pltpu.sync_copy(dsts[r].at[wid, pl.ds(q * g, g)],
                                dst_c.at[pl.ds(0, g)])
                sd = [pltpu.async_copy(ones, acc.at[dst_c.at[j]], sem,
                                       add=True) for j in range(g)]
                for d in sd:
                    d.wait()
                return _

            lax.fori_loop(0, nch, chunk, None)
            plsc.subcore_barrier()
            pltpu.sync_copy(acc.at[pl.ds(tl * rpt, rpt)],
                            outs[oi].at[sc, pl.ds(tl * rpt, rpt),
                                        pl.ds(16 * col, 16)])
            if r + 1 < n:
                plsc.subcore_barrier()

    return counts


def _pad_edges(e, e_pad, n_dst):
    pad = e_pad - e.shape[1]
    src = jnp.concatenate([e[0], jnp.zeros((pad,), jnp.int32)])
    dst = jnp.concatenate([e[1], jnp.full((pad,), n_dst, jnp.int32)])
    return src.reshape(32, -1, 128), dst.reshape(32, -1, 128)


def kernel(x_fm, x_sm, gf, period_vol, edge_qoq, edge_bel, edge_cp, edge_cd,
           edge_rev, params):
    p = params
    pe, wr_sum, bl_sum = _prep(
        p["W_pe"], period_vol, p["Wr_qoq"], p["Wr_cp"], p["Wr_rev"],
        p["bl_qoq"].reshape(1, H), p["bl_cp"].reshape(1, H),
        p["bl_rev"].reshape(1, H))

    xc = tuple(x_fm[:, 32 * c:32 * (c + 1)].astype(jnp.bfloat16)
               for c in range(4))
    pe_bf = pe.astype(jnp.bfloat16)
    edges = {"qoq": edge_qoq, "bel": edge_bel, "cp": edge_cp,
             "cd": edge_cd, "rev": edge_rev}
    ndst = {"qoq": N_FM, "bel": N_TP, "cp": N_FM, "cd": N_SM, "rev": N_FM}
    srcs, dsts = {}, {}
    for r, (e_pad, n_pad, k, g) in _CFG.items():
        srcs[r], dsts[r] = _pad_edges(edges[r], e_pad, ndst[r])

    rels = ["qoq", "bel", "cp", "cd", "rev"]
    slot = {"qoq": (0, 0), "bel": (1, 0), "cp": (0, 1), "cd": (2, 0),
            "rev": (0, 2)}
    cnt_cfg = tuple((_CFG[r][0], _CFG[r][1], _CFG[r][3]) + slot[r]
                    for r in rels)
    cfm, ctp, csm = _make_counts(cnt_cfg)(*[dsts[r] for r in rels])

    sd = _make_agg(*_CFG["cd"])(srcs["cd"], dsts["cd"], pe_bf)
    sb = _make_agg(*_CFG["bel"])(srcs["bel"], dsts["bel"], *xc)
    scp = _make_agg(*_CFG["cp"])(srcs["cp"], dsts["cp"], pe_bf)
    srv = _make_agg(*_CFG["rev"])(srcs["rev"], dsts["rev"], pe_bf)
    sq = _make_agg(*_CFG["qoq"])(srcs["qoq"], dsts["qoq"], *xc)

    fm = _fm_mean(x_fm, sq, cfm, scp, srv, p, wr_sum, bl_sum)
    tp = _tp_mean(pe, sb, ctp, p)
    sm = _sm_mean(x_sm, sd, csm, p)
    out = _head(fm, tp, sm, gf, p)
    return out.reshape(())


# speedup vs baseline: 8.4650x; 1.0437x over previous
import functools

import jax
import jax.numpy as jnp
from jax import lax
from jax.experimental import pallas as pl
from jax.experimental.pallas import tpu as pltpu
from jax.experimental.pallas import tpu_sc as plsc

N_FM = 50000
N_TP = 8192
N_SM = 20000
H = 128

P_FM = 50176
P_TP = 8320
P_SM = 20096


def _prep_body(wpe, pvol, wr_qoq, wr_cp, wr_rev, bl_qoq, bl_cp, bl_rev,
               pe_out, wr_out, bl_out):
    pe_out[...] = wpe[...] * pvol[...]
    wr_out[...] = wr_qoq[...] + wr_cp[...] + wr_rev[...]
    bl_out[...] = bl_qoq[...] + bl_cp[...] + bl_rev[...]


def _prep(wpe, pvol, wr_qoq, wr_cp, wr_rev, bl_qoq, bl_cp, bl_rev):
    return pl.pallas_call(
        _prep_body,
        out_shape=(
            jax.ShapeDtypeStruct((N_TP, 32), jnp.float32),
            jax.ShapeDtypeStruct((H, H), jnp.float32),
            jax.ShapeDtypeStruct((1, H), jnp.float32),
        ),
    )(wpe, pvol, wr_qoq, wr_cp, wr_rev, bl_qoq, bl_cp, bl_rev)


def _ln_relu(o, g, b):
    mu = jnp.mean(o, axis=1, keepdims=True)
    d = o - mu
    v = jnp.mean(d * d, axis=1, keepdims=True)
    y = d * lax.rsqrt(v + 1e-5) * g + b
    return jnp.maximum(y, 0.0)


def _inv16(cnt, lo):
    c = jnp.sum(cnt[:, lo:lo + 16], axis=1, keepdims=True)
    return 1.0 / jnp.maximum(c, 1.0)


def _agg_matmul(sum2, inv, wl_ref):
    return jax.lax.dot_general(sum2 * inv, wl_ref[...],
                               (((1,), (1,)), ((), ())),
                               preferred_element_type=jnp.float32)


def _fm_body(x_ref, sq_ref, cfm_ref, sc_ref, sr_ref,
             wlq_ref, wlc_ref, wlr_ref, wr_ref, bl_ref, g_ref, b_ref,
             out_ref, nblk):
    i = pl.program_id(0)
    cnt = cfm_ref[0] + cfm_ref[1]
    o = _agg_matmul(sq_ref[0].astype(jnp.float32)
                    + sq_ref[1].astype(jnp.float32), _inv16(cnt, 0), wlq_ref)
    o += _agg_matmul((sc_ref[0].astype(jnp.float32)
                      + sc_ref[1].astype(jnp.float32))[:, 0:32],
                     _inv16(cnt, 16), wlc_ref)
    o += _agg_matmul((sr_ref[0].astype(jnp.float32)
                      + sr_ref[1].astype(jnp.float32))[:, 0:32],
                     _inv16(cnt, 32), wlr_ref)
    o += jax.lax.dot_general(x_ref[...], wr_ref[...],
                             (((1,), (1,)), ((), ())),
                             preferred_element_type=jnp.float32)
    o = (o + bl_ref[...]) / 3.0
    y = _ln_relu(o, g_ref[...], b_ref[...])
    s = jnp.sum(y, axis=0, keepdims=True)

    @pl.when(i == 0)
    def _():
        out_ref[...] = jnp.zeros_like(out_ref)
    out_ref[...] += s
    @pl.when(i == nblk - 1)
    def _():
        out_ref[...] = out_ref[...] * (1.0 / N_FM)


def _fm_mean(x_fm, sq, cfm, scp, srv, p, wr_sum, bl_sum):
    blk, nblk = 400, 125
    grid = (nblk,)
    full = lambda *s: pl.BlockSpec(s, lambda i: (0,) * len(s))
    return pl.pallas_call(
        functools.partial(_fm_body, nblk=nblk),
        grid=grid,
        in_specs=[
            pl.BlockSpec((blk, H), lambda i: (i, 0)),
            pl.BlockSpec((2, blk, H), lambda i: (0, i, 0)),
            pl.BlockSpec((2, blk, H), lambda i: (0, i, 0)),
            pl.BlockSpec((2, blk, H), lambda i: (0, i, 0)),
            pl.BlockSpec((2, blk, H), lambda i: (0, i, 0)),
            full(H, H), full(H, 32), full(H, 32), full(H, H),
            full(1, H), full(1, H), full(1, H),
        ],
        out_specs=pl.BlockSpec((1, H), lambda i: (0, 0)),
        out_shape=jax.ShapeDtypeStruct((1, H), jnp.float32),
    )(x_fm, sq, cfm, scp, srv,
      p["Wl_qoq"], p["Wl_cp"], p["Wl_rev"], wr_sum, bl_sum,
      p["g_fm"].reshape(1, H), p["b_fm"].reshape(1, H))


def _tp_body(pe_ref, sb_ref, cb_ref, wlb_ref, wrb_ref, bl_ref, g_ref, b_ref,
             out_ref, nblk):
    i = pl.program_id(0)
    o = _agg_matmul(sb_ref[0].astype(jnp.float32)
                    + sb_ref[1].astype(jnp.float32),
                    _inv16(cb_ref[0] + cb_ref[1], 0), wlb_ref)
    o += jax.lax.dot_general(pe_ref[...], wrb_ref[...],
                             (((1,), (1,)), ((), ())),
                             preferred_element_type=jnp.float32)
    o += bl_ref[...]
    y = _ln_relu(o, g_ref[...], b_ref[...])
    s = jnp.sum(y, axis=0, keepdims=True)

    @pl.when(i == 0)
    def _():
        out_ref[...] = jnp.zeros_like(out_ref)
    out_ref[...] += s
    @pl.when(i == nblk - 1)
    def _():
        out_ref[...] = out_ref[...] * (1.0 / N_TP)


def _tp_mean(pe, sb, cb, p):
    blk, nblk = 512, 16
    full = lambda *s: pl.BlockSpec(s, lambda i: (0,) * len(s))
    return pl.pallas_call(
        functools.partial(_tp_body, nblk=nblk),
        grid=(nblk,),
        in_specs=[
            pl.BlockSpec((blk, 32), lambda i: (i, 0)),
            pl.BlockSpec((2, blk, H), lambda i: (0, i, 0)),
            pl.BlockSpec((2, blk, H), lambda i: (0, i, 0)),
            full(H, H), full(H, 32), full(1, H), full(1, H), full(1, H),
        ],
        out_specs=pl.BlockSpec((1, H), lambda i: (0, 0)),
        out_shape=jax.ShapeDtypeStruct((1, H), jnp.float32),
    )(pe, sb, cb, p["Wl_bel"], p["Wr_bel"], p["bl_bel"].reshape(1, H),
      p["g_tp"].reshape(1, H), p["b_tp"].reshape(1, H))


def _sm_body(x_ref, sd_ref, cd_ref, wld_ref, wrd_ref, bl_ref, g_ref, b_ref,
             out_ref, nblk):
    i = pl.program_id(0)
    o = _agg_matmul((sd_ref[0].astype(jnp.float32)
                     + sd_ref[1].astype(jnp.float32))[:, 0:32],
                    _inv16(cd_ref[0] + cd_ref[1], 0), wld_ref)
    o += jax.lax.dot_general(x_ref[...], wrd_ref[...],
                             (((1,), (1,)), ((), ())),
                             preferred_element_type=jnp.float32)
    o += bl_ref[...]
    y = _ln_relu(o, g_ref[...], b_ref[...])
    s = jnp.sum(y, axis=0, keepdims=True)

    @pl.when(i == 0)
    def _():
        out_ref[...] = jnp.zeros_like(out_ref)
    out_ref[...] += s
    @pl.when(i == nblk - 1)
    def _():
        out_ref[...] = out_ref[...] * (1.0 / N_SM)


def _sm_mean(x_sm, sd, cd, p):
    blk, nblk = 400, 50
    full = lambda *s: pl.BlockSpec(s, lambda i: (0,) * len(s))
    return pl.pallas_call(
        functools.partial(_sm_body, nblk=nblk),
        grid=(nblk,),
        in_specs=[
            pl.BlockSpec((blk, H), lambda i: (i, 0)),
            pl.BlockSpec((2, blk, H), lambda i: (0, i, 0)),
            pl.BlockSpec((2, blk, H), lambda i: (0, i, 0)),
            full(H, 32), full(H, H), full(1, H), full(1, H), full(1, H),
        ],
        out_specs=pl.BlockSpec((1, H), lambda i: (0, 0)),
        out_shape=jax.ShapeDtypeStruct((1, H), jnp.float32),
    )(x_sm, sd, cd, p["Wl_cd"], p["Wr_cd"], p["bl_cd"].reshape(1, H),
      p["g_sm"].reshape(1, H), p["b_sm"].reshape(1, H))


def _head_body(fm_ref, tp_ref, sm_ref, gf_ref, w1_ref, b1_ref, w2_ref,
               b2_ref, out_ref):
    h = jnp.concatenate(
        [fm_ref[...], tp_ref[...], sm_ref[...], gf_ref[...]], axis=1)
    h1 = jax.lax.dot_general(h, w1_ref[...], (((1,), (1,)), ((), ())),
                             preferred_element_type=jnp.float32)
    h1 = jnp.maximum(h1 + b1_ref[...], 0.0)
    out_ref[0, 0] = jnp.sum(h1 * w2_ref[...]) + b2_ref[0, 0]


def _head(fm, tp, sm, gf, p):
    return pl.pallas_call(
        _head_body,
        in_specs=[pl.BlockSpec(memory_space=pltpu.VMEM)] * 7
        + [pl.BlockSpec(memory_space=pltpu.SMEM)],
        out_specs=pl.BlockSpec(memory_space=pltpu.SMEM),
        out_shape=jax.ShapeDtypeStruct((1, 1), jnp.float32),
    )(fm, tp, sm, gf, p["W1"], p["b1"].reshape(1, 64), p["W2"],
      p["b2"].reshape(1, 1))


_SC_PARAMS = pltpu.CompilerParams(use_tc_tiling_on_sc=False)


@functools.lru_cache(maxsize=None)
def _sc_mesh():
    return plsc.VectorSubcoreMesh(core_axis_name="c", subcore_axis_name="s")

_CFG = {
    "qoq": (614400, P_FM, 4, 30),
    "bel": (401408, P_TP, 4, 14),
    "cp": (401408, P_FM, 1, 14),
    "cd": (204800, P_SM, 1, 10),
    "rev": (401408, P_FM, 1, 14),
}


def _zero_vmem(ref, rows, width):
    z = jnp.zeros((16,), jnp.float32)

    def zb(i, _):
        for w in range(width // 16):
            ref[i, pl.ds(w * 16, 16)] = z
        return _

    lax.fori_loop(0, rows, zb, None)


@functools.lru_cache(maxsize=None)
def _make_agg(e_pad, n_pad, k, g):
    ns = e_pad // 32 // 128
    nch = ns // g
    rpt = n_pad // 16
    zn = 16 if rpt % 16 == 0 else 8
    zr = rpt // zn

    nb = 8

    @functools.partial(
        pl.kernel, mesh=_sc_mesh(), compiler_params=_SC_PARAMS,
        out_type=jax.ShapeDtypeStruct((2, n_pad, 128), jnp.bfloat16),
        scratch_types=[
            pltpu.VMEM((g, 128), jnp.int32),
            pltpu.VMEM((g, 128), jnp.int32),
            [pltpu.VMEM((128, 32), jnp.bfloat16)] * nb,
            pltpu.VMEM((zr, 32), jnp.bfloat16),
            pltpu.VMEM_SHARED((n_pad, 32), jnp.bfloat16),
            [pltpu.SemaphoreType.DMA] * nb,
            [pltpu.SemaphoreType.DMA] * nb,
        ],
    )
    def agg(src3d, dst3d, *rest):
        tables = rest[:k]
        out = rest[k]
        src_c, dst_c, bufs, zv, acc, gsems, ssems = rest[k + 1:]
        sc = lax.axis_index("c")
        tl = lax.axis_index("s")
        wid = sc * 16 + tl
        base = tl * rpt

        zb16 = jnp.zeros((32,), jnp.bfloat16)

        def _zb(i, _):
            zv[i, pl.ds(0, 32)] = zb16
            return _

        lax.fori_loop(0, zr, _zb, None)
        for z in range(zn):
            pltpu.sync_copy(zv, acc.at[pl.ds(base + z * zr, zr)])

        for c in range(k):
            plsc.subcore_barrier()
            tab = tables[c]

            def chunk(q, _):
                pltpu.sync_copy(src3d.at[wid, pl.ds(q * g, g)], src_c)
                pltpu.sync_copy(dst3d.at[wid, pl.ds(q * g, g)], dst_c)
                gd, sd = {}, {}
                for j in range(g):
                    b = j % nb
                    if j >= nb:
                        sd[j - nb].wait()
                    gd[j] = pltpu.async_copy(tab.at[src_c.at[j]], bufs[b],
                                             gsems[b])
                    if j >= 1:
                        jj = j - 1
                        gd[jj].wait()
                        sd[jj] = pltpu.async_copy(
                            bufs[jj % nb], acc.at[dst_c.at[jj]],
                            ssems[jj % nb], add=True)
                gd[g - 1].wait()
                sd[g - 1] = pltpu.async_copy(
                    bufs[(g - 1) % nb], acc.at[dst_c.at[g - 1]],
                    ssems[(g - 1) % nb], add=True)
                for jj in range(g - nb, g):
                    sd[jj].wait()
                return _

            lax.fori_loop(0, nch, chunk, None)
            plsc.subcore_barrier()
            pltpu.sync_copy(acc.at[pl.ds(base, rpt)],
                            out.at[sc, pl.ds(base, rpt),
                                   pl.ds(32 * c, 32)])
            if c + 1 < k:
                for z in range(zn):
                    pltpu.sync_copy(zv, acc.at[pl.ds(base + z * zr, zr)])

    return agg


@functools.lru_cache(maxsize=None)
def _make_counts(cfgs):
    out_npads = {}
    for _, np_, _, oi, _ in cfgs:
        out_npads[oi] = np_
    n_out = len(out_npads)
    max_np = max(c[1] for c in cfgs)
    max_g = max(c[2] for c in cfgs)
    rpt_max = max_np // 16
    zr = rpt_max // 8

    @functools.partial(
        pl.kernel, mesh=_sc_mesh(), compiler_params=_SC_PARAMS,
        out_type=tuple(jax.ShapeDtypeStruct((2, out_npads[i], 128),
                                            jnp.float32)
                       for i in range(n_out)),
        scratch_types=[
            pltpu.VMEM((max_g, 128), jnp.int32),
            pltpu.VMEM((128, 16), jnp.float32),
            pltpu.VMEM((zr, 16), jnp.float32),
            pltpu.VMEM_SHARED((max_np, 16), jnp.float32),
            pltpu.SemaphoreType.DMA,
        ],
    )
    def counts(*args):
        n = len(cfgs)
        dsts = args[:n]
        outs = args[n:n + n_out]
        dst_c, ones, zv, acc, sem = args[n + n_out:]
        sc = lax.axis_index("c")
        tl = lax.axis_index("s")
        wid = sc * 16 + tl

        _zero_vmem(zv, zr, 16)
        one = jnp.full((16,), 1.0 / 16.0, jnp.float32)

        def ob(i, _):
            ones[i, pl.ds(0, 16)] = one
            return _

        lax.fori_loop(0, 128, ob, None)

        for r, (e_pad, np_, g, oi, col) in enumerate(cfgs):
            ns = e_pad // 32 // 128
            nch = ns // g
            rpt = np_ // 16
            for z in range(8):
                pltpu.sync_copy(zv, acc.at[pl.ds(tl * rpt_max + z * zr, zr)])
            plsc.subcore_barrier()

            def chunk(q, _):
                pltpu.sync_copy(dsts[r].at[wid, pl.ds(q * g, g)],
                                dst_c.at[pl.ds(0, g)])
                sd = [pltpu.async_copy(ones, acc.at[dst_c.at[j]], sem,
                                       add=True) for j in range(g)]
                for d in sd:
                    d.wait()
                return _

            lax.fori_loop(0, nch, chunk, None)
            plsc.subcore_barrier()
            pltpu.sync_copy(acc.at[pl.ds(tl * rpt, rpt)],
                            outs[oi].at[sc, pl.ds(tl * rpt, rpt),
                                        pl.ds(16 * col, 16)])
            if r + 1 < n:
                plsc.subcore_barrier()

    return counts


def _pad_edges(e, e_pad, n_dst):
    pad = e_pad - e.shape[1]
    src = jnp.concatenate([e[0], jnp.zeros((pad,), jnp.int32)])
    dst = jnp.concatenate([e[1], jnp.full((pad,), n_dst, jnp.int32)])
    return src.reshape(32, -1, 128), dst.reshape(32, -1, 128)


def kernel(x_fm, x_sm, gf, period_vol, edge_qoq, edge_bel, edge_cp, edge_cd,
           edge_rev, params):
    p = params
    pe, wr_sum, bl_sum = _prep(
        p["W_pe"], period_vol, p["Wr_qoq"], p["Wr_cp"], p["Wr_rev"],
        p["bl_qoq"].reshape(1, H), p["bl_cp"].reshape(1, H),
        p["bl_rev"].reshape(1, H))

    xc = tuple(x_fm[:, 32 * c:32 * (c + 1)].astype(jnp.bfloat16)
               for c in range(4))
    pe_bf = pe.astype(jnp.bfloat16)
    edges = {"qoq": edge_qoq, "bel": edge_bel, "cp": edge_cp,
             "cd": edge_cd, "rev": edge_rev}
    ndst = {"qoq": N_FM, "bel": N_TP, "cp": N_FM, "cd": N_SM, "rev": N_FM}
    srcs, dsts = {}, {}
    for r, (e_pad, n_pad, k, g) in _CFG.items():
        srcs[r], dsts[r] = _pad_edges(edges[r], e_pad, ndst[r])

    rels = ["qoq", "bel", "cp", "cd", "rev"]
    slot = {"qoq": (0, 0), "bel": (1, 0), "cp": (0, 1), "cd": (2, 0),
            "rev": (0, 2)}
    cnt_cfg = tuple((_CFG[r][0], _CFG[r][1], _CFG[r][3]) + slot[r]
                    for r in rels)
    cfm, ctp, csm = _make_counts(cnt_cfg)(*[dsts[r] for r in rels])

    sd = _make_agg(*_CFG["cd"])(srcs["cd"], dsts["cd"], pe_bf)
    sb = _make_agg(*_CFG["bel"])(srcs["bel"], dsts["bel"], *xc)
    scp = _make_agg(*_CFG["cp"])(srcs["cp"], dsts["cp"], pe_bf)
    srv = _make_agg(*_CFG["rev"])(srcs["rev"], dsts["rev"], pe_bf)
    sq = _make_agg(*_CFG["qoq"])(srcs["qoq"], dsts["qoq"], *xc)

    fm = _fm_mean(x_fm, sq, cfm, scp, srv, p, wr_sum, bl_sum)
    tp = _tp_mean(pe, sb, ctp, p)
    sm = _sm_mean(x_sm, sd, csm, p)
    out = _head(fm, tp, sm, gf, p)
    return out.reshape(())


# speedup vs baseline: 8.7500x; 1.0337x over previous
import functools

import jax
import jax.numpy as jnp
from jax import lax
from jax.experimental import pallas as pl
from jax.experimental.pallas import tpu as pltpu
from jax.experimental.pallas import tpu_sc as plsc

N_FM = 50000
N_TP = 8192
N_SM = 20000
H = 128

P_FM = 50176
P_TP = 8320
P_SM = 20096


def _prep_body(wpe, pvol, wr_qoq, wr_cp, wr_rev, bl_qoq, bl_cp, bl_rev,
               pe_out, wr_out, bl_out):
    pe_out[...] = wpe[...] * pvol[...]
    wr_out[...] = wr_qoq[...] + wr_cp[...] + wr_rev[...]
    bl_out[...] = bl_qoq[...] + bl_cp[...] + bl_rev[...]


def _prep(wpe, pvol, wr_qoq, wr_cp, wr_rev, bl_qoq, bl_cp, bl_rev):
    return pl.pallas_call(
        _prep_body,
        out_shape=(
            jax.ShapeDtypeStruct((N_TP, 32), jnp.float32),
            jax.ShapeDtypeStruct((H, H), jnp.float32),
            jax.ShapeDtypeStruct((1, H), jnp.float32),
        ),
    )(wpe, pvol, wr_qoq, wr_cp, wr_rev, bl_qoq, bl_cp, bl_rev)


def _ln_relu(o, g, b):
    mu = jnp.mean(o, axis=1, keepdims=True)
    d = o - mu
    v = jnp.mean(d * d, axis=1, keepdims=True)
    y = d * lax.rsqrt(v + 1e-5) * g + b
    return jnp.maximum(y, 0.0)


def _inv16(cnt, lo):
    c = jnp.sum(cnt[:, lo:lo + 16], axis=1, keepdims=True)
    return 1.0 / jnp.maximum(c, 1.0)


def _agg_matmul(sum2, inv, wl_ref):
    return jax.lax.dot_general(sum2 * inv, wl_ref[...],
                               (((1,), (1,)), ((), ())),
                               preferred_element_type=jnp.float32)


def _fm_body(x_ref, sq_ref, cfm_ref, sc_ref, sr_ref,
             wlq_ref, wlc_ref, wlr_ref, wr_ref, bl_ref, g_ref, b_ref,
             out_ref, nblk):
    i = pl.program_id(0)
    cnt = cfm_ref[0] + cfm_ref[1]
    o = _agg_matmul(sq_ref[0].astype(jnp.float32)
                    + sq_ref[1].astype(jnp.float32), _inv16(cnt, 0), wlq_ref)
    o += _agg_matmul((sc_ref[0].astype(jnp.float32)
                      + sc_ref[1].astype(jnp.float32))[:, 0:32],
                     _inv16(cnt, 16), wlc_ref)
    o += _agg_matmul((sr_ref[0].astype(jnp.float32)
                      + sr_ref[1].astype(jnp.float32))[:, 0:32],
                     _inv16(cnt, 32), wlr_ref)
    o += jax.lax.dot_general(x_ref[...], wr_ref[...],
                             (((1,), (1,)), ((), ())),
                             preferred_element_type=jnp.float32)
    o = (o + bl_ref[...]) / 3.0
    y = _ln_relu(o, g_ref[...], b_ref[...])
    s = jnp.sum(y, axis=0, keepdims=True)

    @pl.when(i == 0)
    def _():
        out_ref[...] = jnp.zeros_like(out_ref)
    out_ref[...] += s
    @pl.when(i == nblk - 1)
    def _():
        out_ref[...] = out_ref[...] * (1.0 / N_FM)


def _fm_mean(x_fm, sq, cfm, scp, srv, p, wr_sum, bl_sum):
    blk, nblk = 400, 125
    grid = (nblk,)
    full = lambda *s: pl.BlockSpec(s, lambda i: (0,) * len(s))
    return pl.pallas_call(
        functools.partial(_fm_body, nblk=nblk),
        grid=grid,
        in_specs=[
            pl.BlockSpec((blk, H), lambda i: (i, 0)),
            pl.BlockSpec((2, blk, H), lambda i: (0, i, 0)),
            pl.BlockSpec((2, blk, H), lambda i: (0, i, 0)),
            pl.BlockSpec((2, blk, H), lambda i: (0, i, 0)),
            pl.BlockSpec((2, blk, H), lambda i: (0, i, 0)),
            full(H, H), full(H, 32), full(H, 32), full(H, H),
            full(1, H), full(1, H), full(1, H),
        ],
        out_specs=pl.BlockSpec((1, H), lambda i: (0, 0)),
        out_shape=jax.ShapeDtypeStruct((1, H), jnp.float32),
    )(x_fm, sq, cfm, scp, srv,
      p["Wl_qoq"], p["Wl_cp"], p["Wl_rev"], wr_sum, bl_sum,
      p["g_fm"].reshape(1, H), p["b_fm"].reshape(1, H))


def _tp_body(pe_ref, sb_ref, cb_ref, wlb_ref, wrb_ref, bl_ref, g_ref, b_ref,
             out_ref, nblk):
    i = pl.program_id(0)
    o = _agg_matmul(sb_ref[0].astype(jnp.float32)
                    + sb_ref[1].astype(jnp.float32),
                    _inv16(cb_ref[0] + cb_ref[1], 0), wlb_ref)
    o += jax.lax.dot_general(pe_ref[...], wrb_ref[...],
                             (((1,), (1,)), ((), ())),
                             preferred_element_type=jnp.float32)
    o += bl_ref[...]
    y = _ln_relu(o, g_ref[...], b_ref[...])
    s = jnp.sum(y, axis=0, keepdims=True)

    @pl.when(i == 0)
    def _():
        out_ref[...] = jnp.zeros_like(out_ref)
    out_ref[...] += s
    @pl.when(i == nblk - 1)
    def _():
        out_ref[...] = out_ref[...] * (1.0 / N_TP)


def _tp_mean(pe, sb, cb, p):
    blk, nblk = 512, 16
    full = lambda *s: pl.BlockSpec(s, lambda i: (0,) * len(s))
    return pl.pallas_call(
        functools.partial(_tp_body, nblk=nblk),
        grid=(nblk,),
        in_specs=[
            pl.BlockSpec((blk, 32), lambda i: (i, 0)),
            pl.BlockSpec((2, blk, H), lambda i: (0, i, 0)),
            pl.BlockSpec((2, blk, H), lambda i: (0, i, 0)),
            full(H, H), full(H, 32), full(1, H), full(1, H), full(1, H),
        ],
        out_specs=pl.BlockSpec((1, H), lambda i: (0, 0)),
        out_shape=jax.ShapeDtypeStruct((1, H), jnp.float32),
    )(pe, sb, cb, p["Wl_bel"], p["Wr_bel"], p["bl_bel"].reshape(1, H),
      p["g_tp"].reshape(1, H), p["b_tp"].reshape(1, H))


def _sm_body(x_ref, sd_ref, cd_ref, wld_ref, wrd_ref, bl_ref, g_ref, b_ref,
             out_ref, nblk):
    i = pl.program_id(0)
    o = _agg_matmul((sd_ref[0].astype(jnp.float32)
                     + sd_ref[1].astype(jnp.float32))[:, 0:32],
                    _inv16(cd_ref[0] + cd_ref[1], 0), wld_ref)
    o += jax.lax.dot_general(x_ref[...], wrd_ref[...],
                             (((1,), (1,)), ((), ())),
                             preferred_element_type=jnp.float32)
    o += bl_ref[...]
    y = _ln_relu(o, g_ref[...], b_ref[...])
    s = jnp.sum(y, axis=0, keepdims=True)

    @pl.when(i == 0)
    def _():
        out_ref[...] = jnp.zeros_like(out_ref)
    out_ref[...] += s
    @pl.when(i == nblk - 1)
    def _():
        out_ref[...] = out_ref[...] * (1.0 / N_SM)


def _sm_mean(x_sm, sd, cd, p):
    blk, nblk = 400, 50
    full = lambda *s: pl.BlockSpec(s, lambda i: (0,) * len(s))
    return pl.pallas_call(
        functools.partial(_sm_body, nblk=nblk),
        grid=(nblk,),
        in_specs=[
            pl.BlockSpec((blk, H), lambda i: (i, 0)),
            pl.BlockSpec((2, blk, H), lambda i: (0, i, 0)),
            pl.BlockSpec((2, blk, H), lambda i: (0, i, 0)),
            full(H, 32), full(H, H), full(1, H), full(1, H), full(1, H),
        ],
        out_specs=pl.BlockSpec((1, H), lambda i: (0, 0)),
        out_shape=jax.ShapeDtypeStruct((1, H), jnp.float32),
    )(x_sm, sd, cd, p["Wl_cd"], p["Wr_cd"], p["bl_cd"].reshape(1, H),
      p["g_sm"].reshape(1, H), p["b_sm"].reshape(1, H))


def _head_body(fm_ref, tp_ref, sm_ref, gf_ref, w1_ref, b1_ref, w2_ref,
               b2_ref, out_ref):
    h = jnp.concatenate(
        [fm_ref[...], tp_ref[...], sm_ref[...], gf_ref[...]], axis=1)
    h1 = jax.lax.dot_general(h, w1_ref[...], (((1,), (1,)), ((), ())),
                             preferred_element_type=jnp.float32)
    h1 = jnp.maximum(h1 + b1_ref[...], 0.0)
    out_ref[0, 0] = jnp.sum(h1 * w2_ref[...]) + b2_ref[0, 0]


def _head(fm, tp, sm, gf, p):
    return pl.pallas_call(
        _head_body,
        in_specs=[pl.BlockSpec(memory_space=pltpu.VMEM)] * 7
        + [pl.BlockSpec(memory_space=pltpu.SMEM)],
        out_specs=pl.BlockSpec(memory_space=pltpu.SMEM),
        out_shape=jax.ShapeDtypeStruct((1, 1), jnp.float32),
    )(fm, tp, sm, gf, p["W1"], p["b1"].reshape(1, 64), p["W2"],
      p["b2"].reshape(1, 1))


_SC_PARAMS = pltpu.CompilerParams(use_tc_tiling_on_sc=False)


@functools.lru_cache(maxsize=None)
def _sc_mesh():
    return plsc.VectorSubcoreMesh(core_axis_name="c", subcore_axis_name="s")

_CFG = {
    "qoq": (614400, P_FM, 4, 30),
    "bel": (401408, P_TP, 4, 49),
    "cp": (401408, P_FM, 1, 49),
    "cd": (204800, P_SM, 1, 25),
    "rev": (401408, P_FM, 1, 49),
}


def _zero_vmem(ref, rows, width):
    z = jnp.zeros((16,), jnp.float32)

    def zb(i, _):
        for w in range(width // 16):
            ref[i, pl.ds(w * 16, 16)] = z
        return _

    lax.fori_loop(0, rows, zb, None)


@functools.lru_cache(maxsize=None)
def _make_agg(e_pad, n_pad, k, g):
    ns = e_pad // 32 // 128
    nch = ns // g
    rpt = n_pad // 16
    zn = 16 if rpt % 16 == 0 else 8
    zr = rpt // zn

    nb = 8

    @functools.partial(
        pl.kernel, mesh=_sc_mesh(), compiler_params=_SC_PARAMS,
        out_type=jax.ShapeDtypeStruct((2, n_pad, 128), jnp.bfloat16),
        scratch_types=[
            pltpu.VMEM((g, 128), jnp.int32),
            pltpu.VMEM((g, 128), jnp.int32),
            [pltpu.VMEM((128, 32), jnp.bfloat16)] * nb,
            pltpu.VMEM((zr, 32), jnp.bfloat16),
            pltpu.VMEM_SHARED((n_pad, 32), jnp.bfloat16),
            [pltpu.SemaphoreType.DMA] * nb,
            [pltpu.SemaphoreType.DMA] * nb,
        ],
    )
    def agg(src3d, dst3d, *rest):
        tables = rest[:k]
        out = rest[k]
        src_c, dst_c, bufs, zv, acc, gsems, ssems = rest[k + 1:]
        sc = lax.axis_index("c")
        tl = lax.axis_index("s")
        wid = sc * 16 + tl
        base = tl * rpt

        zb16 = jnp.zeros((32,), jnp.bfloat16)

        def _zb(i, _):
            zv[i, pl.ds(0, 32)] = zb16
            return _

        lax.fori_loop(0, zr, _zb, None)
        for z in range(zn):
            pltpu.sync_copy(zv, acc.at[pl.ds(base + z * zr, zr)])

        for c in range(k):
            plsc.subcore_barrier()
            tab = tables[c]

            def chunk(q, _):
                pltpu.sync_copy(src3d.at[wid, pl.ds(q * g, g)], src_c)
                pltpu.sync_copy(dst3d.at[wid, pl.ds(q * g, g)], dst_c)
                gd, sd = {}, {}
                for j in range(g):
                    b = j % nb
                    if j >= nb:
                        sd[j - nb].wait()
                    gd[j] = pltpu.async_copy(tab.at[src_c.at[j]], bufs[b],
                                             gsems[b])
                    if j >= 1:
                        jj = j - 1
                        gd[jj].wait()
                        sd[jj] = pltpu.async_copy(
                            bufs[jj % nb], acc.at[dst_c.at[jj]],
                            ssems[jj % nb], add=True)
                gd[g - 1].wait()
                sd[g - 1] = pltpu.async_copy(
                    bufs[(g - 1) % nb], acc.at[dst_c.at[g - 1]],
                    ssems[(g - 1) % nb], add=True)
                for jj in range(g - nb, g):
                    sd[jj].wait()
                return _

            lax.fori_loop(0, nch, chunk, None)
            plsc.subcore_barrier()
            pltpu.sync_copy(acc.at[pl.ds(base, rpt)],
                            out.at[sc, pl.ds(base, rpt),
                                   pl.ds(32 * c, 32)])
            if c + 1 < k:
                for z in range(zn):
                    pltpu.sync_copy(zv, acc.at[pl.ds(base + z * zr, zr)])

    return agg


@functools.lru_cache(maxsize=None)
def _make_counts(cfgs):
    out_npads = {}
    for _, np_, _, oi, _ in cfgs:
        out_npads[oi] = np_
    n_out = len(out_npads)
    max_np = max(c[1] for c in cfgs)
    max_g = max(c[2] for c in cfgs)
    rpt_max = max_np // 16
    zr = rpt_max // 8

    @functools.partial(
        pl.kernel, mesh=_sc_mesh(), compiler_params=_SC_PARAMS,
        out_type=tuple(jax.ShapeDtypeStruct((2, out_npads[i], 128),
                                            jnp.float32)
                       for i in range(n_out)),
        scratch_types=[
            pltpu.VMEM((max_g, 128), jnp.int32),
            pltpu.VMEM((128, 16), jnp.float32),
            pltpu.VMEM((zr, 16), jnp.float32),
            pltpu.VMEM_SHARED((max_np, 16), jnp.float32),
            pltpu.SemaphoreType.DMA,
        ],
    )
    def counts(*args):
        n = len(cfgs)
        dsts = args[:n]
        outs = args[n:n + n_out]
        dst_c, ones, zv, acc, sem = args[n + n_out:]
        sc = lax.axis_index("c")
        tl = lax.axis_index("s")
        wid = sc * 16 + tl

        _zero_vmem(zv, zr, 16)
        one = jnp.full((16,), 1.0 / 16.0, jnp.float32)

        def ob(i, _):
            ones[i, pl.ds(0, 16)] = one
            return _

        lax.fori_loop(0, 128, ob, None)

        for r, (e_pad, np_, g, oi, col) in enumerate(cfgs):
            ns = e_pad // 32 // 128
            nch = ns // g
            rpt = np_ // 16
            for z in range(8):
                pltpu.sync_copy(zv, acc.at[pl.ds(tl * rpt_max + z * zr, zr)])
            plsc.subcore_barrier()

            def chunk(q, _):
                pltpu.sync_copy(dsts[r].at[wid, pl.ds(q * g, g)],
                                dst_c.at[pl.ds(0, g)])
                sd = [pltpu.async_copy(ones, acc.at[dst_c.at[j]], sem,
                                       add=True) for j in range(g)]
                for d in sd:
                    d.wait()
                return _

            lax.fori_loop(0, nch, chunk, None)
            plsc.subcore_barrier()
            pltpu.sync_copy(acc.at[pl.ds(tl * rpt, rpt)],
                            outs[oi].at[sc, pl.ds(tl * rpt, rpt),
                                        pl.ds(16 * col, 16)])
            if r + 1 < n:
                plsc.subcore_barrier()

    return counts


def _pad_edges(e, e_pad, n_dst):
    pad = e_pad - e.shape[1]
    src = jnp.concatenate([e[0], jnp.zeros((pad,), jnp.int32)])
    dst = jnp.concatenate([e[1], jnp.full((pad,), n_dst, jnp.int32)])
    return src.reshape(32, -1, 128), dst.reshape(32, -1, 128)


def kernel(x_fm, x_sm, gf, period_vol, edge_qoq, edge_bel, edge_cp, edge_cd,
           edge_rev, params):
    p = params
    pe, wr_sum, bl_sum = _prep(
        p["W_pe"], period_vol, p["Wr_qoq"], p["Wr_cp"], p["Wr_rev"],
        p["bl_qoq"].reshape(1, H), p["bl_cp"].reshape(1, H),
        p["bl_rev"].reshape(1, H))

    xc = tuple(x_fm[:, 32 * c:32 * (c + 1)].astype(jnp.bfloat16)
               for c in range(4))
    pe_bf = pe.astype(jnp.bfloat16)
    edges = {"qoq": edge_qoq, "bel": edge_bel, "cp": edge_cp,
             "cd": edge_cd, "rev": edge_rev}
    ndst = {"qoq": N_FM, "bel": N_TP, "cp": N_FM, "cd": N_SM, "rev": N_FM}
    srcs, dsts = {}, {}
    for r, (e_pad, n_pad, k, g) in _CFG.items():
        srcs[r], dsts[r] = _pad_edges(edges[r], e_pad, ndst[r])

    rels = ["qoq", "bel", "cp", "cd", "rev"]
    slot = {"qoq": (0, 0), "bel": (1, 0), "cp": (0, 1), "cd": (2, 0),
            "rev": (0, 2)}
    cnt_cfg = tuple((_CFG[r][0], _CFG[r][1], _CFG[r][3]) + slot[r]
                    for r in rels)
    cfm, ctp, csm = _make_counts(cnt_cfg)(*[dsts[r] for r in rels])

    sd = _make_agg(*_CFG["cd"])(srcs["cd"], dsts["cd"], pe_bf)
    sb = _make_agg(*_CFG["bel"])(srcs["bel"], dsts["bel"], *xc)
    scp = _make_agg(*_CFG["cp"])(srcs["cp"], dsts["cp"], pe_bf)
    srv = _make_agg(*_CFG["rev"])(srcs["rev"], dsts["rev"], pe_bf)
    sq = _make_agg(*_CFG["qoq"])(srcs["qoq"], dsts["qoq"], *xc)

    fm = _fm_mean(x_fm, sq, cfm, scp, srv, p, wr_sum, bl_sum)
    tp = _tp_mean(pe, sb, ctp, p)
    sm = _sm_mean(x_sm, sd, csm, p)
    out = _head(fm, tp, sm, gf, p)
    return out.reshape(())


# speedup vs baseline: 8.8366x; 1.0099x over previous
import functools

import jax
import jax.numpy as jnp
from jax import lax
from jax.experimental import pallas as pl
from jax.experimental.pallas import tpu as pltpu
from jax.experimental.pallas import tpu_sc as plsc

N_FM = 50000
N_TP = 8192
N_SM = 20000
H = 128

P_FM = 50176
P_TP = 8320
P_SM = 20096


def _prep_body(wpe, pvol, wr_qoq, wr_cp, wr_rev, bl_qoq, bl_cp, bl_rev,
               pe_out, wr_out, bl_out):
    pe_out[...] = wpe[...] * pvol[...]
    wr_out[...] = wr_qoq[...] + wr_cp[...] + wr_rev[...]
    bl_out[...] = bl_qoq[...] + bl_cp[...] + bl_rev[...]


def _prep(wpe, pvol, wr_qoq, wr_cp, wr_rev, bl_qoq, bl_cp, bl_rev):
    return pl.pallas_call(
        _prep_body,
        out_shape=(
            jax.ShapeDtypeStruct((N_TP, 32), jnp.float32),
            jax.ShapeDtypeStruct((H, H), jnp.float32),
            jax.ShapeDtypeStruct((1, H), jnp.float32),
        ),
    )(wpe, pvol, wr_qoq, wr_cp, wr_rev, bl_qoq, bl_cp, bl_rev)


def _ln_relu(o, g, b):
    mu = jnp.mean(o, axis=1, keepdims=True)
    d = o - mu
    v = jnp.mean(d * d, axis=1, keepdims=True)
    y = d * lax.rsqrt(v + 1e-5) * g + b
    return jnp.maximum(y, 0.0)


def _inv16(cnt, lo):
    c = jnp.sum(cnt[:, lo:lo + 16], axis=1, keepdims=True)
    return 1.0 / jnp.maximum(c, 1.0)


def _agg_matmul(sum2, inv, wl_ref):
    return jax.lax.dot_general(sum2 * inv, wl_ref[...],
                               (((1,), (1,)), ((), ())),
                               preferred_element_type=jnp.float32)


def _fm_body(x_ref, sq_ref, cfm_ref, sc_ref, sr_ref,
             wlq_ref, wlc_ref, wlr_ref, wr_ref, bl_ref, g_ref, b_ref,
             out_ref, nblk):
    i = pl.program_id(0)
    cnt = cfm_ref[0] + cfm_ref[1]
    o = _agg_matmul(sq_ref[0].astype(jnp.float32)
                    + sq_ref[1].astype(jnp.float32), _inv16(cnt, 0), wlq_ref)
    o += _agg_matmul((sc_ref[0].astype(jnp.float32)
                      + sc_ref[1].astype(jnp.float32))[:, 0:32],
                     _inv16(cnt, 16), wlc_ref)
    o += _agg_matmul((sr_ref[0].astype(jnp.float32)
                      + sr_ref[1].astype(jnp.float32))[:, 0:32],
                     _inv16(cnt, 32), wlr_ref)
    o += jax.lax.dot_general(x_ref[...], wr_ref[...],
                             (((1,), (1,)), ((), ())),
                             preferred_element_type=jnp.float32)
    o = (o + bl_ref[...]) / 3.0
    y = _ln_relu(o, g_ref[...], b_ref[...])
    s = jnp.sum(y, axis=0, keepdims=True)

    @pl.when(i == 0)
    def _():
        out_ref[...] = jnp.zeros_like(out_ref)
    out_ref[...] += s
    @pl.when(i == nblk - 1)
    def _():
        out_ref[...] = out_ref[...] * (1.0 / N_FM)


def _fm_mean(x_fm, sq, cfm, scp, srv, p, wr_sum, bl_sum):
    blk, nblk = 400, 125
    grid = (nblk,)
    full = lambda *s: pl.BlockSpec(s, lambda i: (0,) * len(s))
    return pl.pallas_call(
        functools.partial(_fm_body, nblk=nblk),
        grid=grid,
        in_specs=[
            pl.BlockSpec((blk, H), lambda i: (i, 0)),
            pl.BlockSpec((2, blk, H), lambda i: (0, i, 0)),
            pl.BlockSpec((2, blk, H), lambda i: (0, i, 0)),
            pl.BlockSpec((2, blk, H), lambda i: (0, i, 0)),
            pl.BlockSpec((2, blk, H), lambda i: (0, i, 0)),
            full(H, H), full(H, 32), full(H, 32), full(H, H),
            full(1, H), full(1, H), full(1, H),
        ],
        out_specs=pl.BlockSpec((1, H), lambda i: (0, 0)),
        out_shape=jax.ShapeDtypeStruct((1, H), jnp.float32),
    )(x_fm, sq, cfm, scp, srv,
      p["Wl_qoq"], p["Wl_cp"], p["Wl_rev"], wr_sum, bl_sum,
      p["g_fm"].reshape(1, H), p["b_fm"].reshape(1, H))


def _tp_body(pe_ref, sb_ref, cb_ref, wlb_ref, wrb_ref, bl_ref, g_ref, b_ref,
             out_ref, nblk):
    i = pl.program_id(0)
    o = _agg_matmul(sb_ref[0].astype(jnp.float32)
                    + sb_ref[1].astype(jnp.float32),
                    _inv16(cb_ref[0] + cb_ref[1], 0), wlb_ref)
    o += jax.lax.dot_general(pe_ref[...], wrb_ref[...],
                             (((1,), (1,)), ((), ())),
                             preferred_element_type=jnp.float32)
    o += bl_ref[...]
    y = _ln_relu(o, g_ref[...], b_ref[...])
    s = jnp.sum(y, axis=0, keepdims=True)

    @pl.when(i == 0)
    def _():
        out_ref[...] = jnp.zeros_like(out_ref)
    out_ref[...] += s
    @pl.when(i == nblk - 1)
    def _():
        out_ref[...] = out_ref[...] * (1.0 / N_TP)


def _tp_mean(pe, sb, cb, p):
    blk, nblk = 512, 16
    full = lambda *s: pl.BlockSpec(s, lambda i: (0,) * len(s))
    return pl.pallas_call(
        functools.partial(_tp_body, nblk=nblk),
        grid=(nblk,),
        in_specs=[
            pl.BlockSpec((blk, 32), lambda i: (i, 0)),
            pl.BlockSpec((2, blk, H), lambda i: (0, i, 0)),
            pl.BlockSpec((2, blk, H), lambda i: (0, i, 0)),
            full(H, H), full(H, 32), full(1, H), full(1, H), full(1, H),
        ],
        out_specs=pl.BlockSpec((1, H), lambda i: (0, 0)),
        out_shape=jax.ShapeDtypeStruct((1, H), jnp.float32),
    )(pe, sb, cb, p["Wl_bel"], p["Wr_bel"], p["bl_bel"].reshape(1, H),
      p["g_tp"].reshape(1, H), p["b_tp"].reshape(1, H))


def _sm_body(x_ref, sd_ref, cd_ref, wld_ref, wrd_ref, bl_ref, g_ref, b_ref,
             out_ref, nblk):
    i = pl.program_id(0)
    o = _agg_matmul((sd_ref[0].astype(jnp.float32)
                     + sd_ref[1].astype(jnp.float32))[:, 0:32],
                    _inv16(cd_ref[0] + cd_ref[1], 0), wld_ref)
    o += jax.lax.dot_general(x_ref[...], wrd_ref[...],
                             (((1,), (1,)), ((), ())),
                             preferred_element_type=jnp.float32)
    o += bl_ref[...]
    y = _ln_relu(o, g_ref[...], b_ref[...])
    s = jnp.sum(y, axis=0, keepdims=True)

    @pl.when(i == 0)
    def _():
        out_ref[...] = jnp.zeros_like(out_ref)
    out_ref[...] += s
    @pl.when(i == nblk - 1)
    def _():
        out_ref[...] = out_ref[...] * (1.0 / N_SM)


def _sm_mean(x_sm, sd, cd, p):
    blk, nblk = 400, 50
    full = lambda *s: pl.BlockSpec(s, lambda i: (0,) * len(s))
    return pl.pallas_call(
        functools.partial(_sm_body, nblk=nblk),
        grid=(nblk,),
        in_specs=[
            pl.BlockSpec((blk, H), lambda i: (i, 0)),
            pl.BlockSpec((2, blk, H), lambda i: (0, i, 0)),
            pl.BlockSpec((2, blk, H), lambda i: (0, i, 0)),
            full(H, 32), full(H, H), full(1, H), full(1, H), full(1, H),
        ],
        out_specs=pl.BlockSpec((1, H), lambda i: (0, 0)),
        out_shape=jax.ShapeDtypeStruct((1, H), jnp.float32),
    )(x_sm, sd, cd, p["Wl_cd"], p["Wr_cd"], p["bl_cd"].reshape(1, H),
      p["g_sm"].reshape(1, H), p["b_sm"].reshape(1, H))


def _head_body(fm_ref, tp_ref, sm_ref, gf_ref, w1_ref, b1_ref, w2_ref,
               b2_ref, out_ref):
    h = jnp.concatenate(
        [fm_ref[...], tp_ref[...], sm_ref[...], gf_ref[...]], axis=1)
    h1 = jax.lax.dot_general(h, w1_ref[...], (((1,), (1,)), ((), ())),
                             preferred_element_type=jnp.float32)
    h1 = jnp.maximum(h1 + b1_ref[...], 0.0)
    out_ref[0, 0] = jnp.sum(h1 * w2_ref[...]) + b2_ref[0, 0]


def _head(fm, tp, sm, gf, p):
    return pl.pallas_call(
        _head_body,
        in_specs=[pl.BlockSpec(memory_space=pltpu.VMEM)] * 7
        + [pl.BlockSpec(memory_space=pltpu.SMEM)],
        out_specs=pl.BlockSpec(memory_space=pltpu.SMEM),
        out_shape=jax.ShapeDtypeStruct((1, 1), jnp.float32),
    )(fm, tp, sm, gf, p["W1"], p["b1"].reshape(1, 64), p["W2"],
      p["b2"].reshape(1, 1))


_SC_PARAMS = pltpu.CompilerParams(use_tc_tiling_on_sc=False)


@functools.lru_cache(maxsize=None)
def _sc_mesh():
    return plsc.VectorSubcoreMesh(core_axis_name="c", subcore_axis_name="s")

_CFG = {
    "qoq": (614400, P_FM, 4, 50),
    "bel": (401408, P_TP, 4, 49),
    "cp": (401408, P_FM, 1, 49),
    "cd": (204800, P_SM, 1, 25),
    "rev": (401408, P_FM, 1, 49),
}


def _zero_vmem(ref, rows, width):
    z = jnp.zeros((16,), jnp.float32)

    def zb(i, _):
        for w in range(width // 16):
            ref[i, pl.ds(w * 16, 16)] = z
        return _

    lax.fori_loop(0, rows, zb, None)


@functools.lru_cache(maxsize=None)
def _make_agg(e_pad, n_pad, k, g):
    ns = e_pad // 32 // 128
    nch = ns // g
    rpt = n_pad // 16
    zn = 16 if rpt % 16 == 0 else 8
    zr = rpt // zn

    nb = 8

    @functools.partial(
        pl.kernel, mesh=_sc_mesh(), compiler_params=_SC_PARAMS,
        out_type=jax.ShapeDtypeStruct((2, n_pad, 128), jnp.bfloat16),
        scratch_types=[
            pltpu.VMEM((g, 128), jnp.int32),
            pltpu.VMEM((g, 128), jnp.int32),
            [pltpu.VMEM((128, 32), jnp.bfloat16)] * nb,
            pltpu.VMEM((zr, 32), jnp.bfloat16),
            pltpu.VMEM_SHARED((n_pad, 32), jnp.bfloat16),
            [pltpu.SemaphoreType.DMA] * nb,
            [pltpu.SemaphoreType.DMA] * nb,
        ],
    )
    def agg(src3d, dst3d, *rest):
        tables = rest[:k]
        out = rest[k]
        src_c, dst_c, bufs, zv, acc, gsems, ssems = rest[k + 1:]
        sc = lax.axis_index("c")
        tl = lax.axis_index("s")
        wid = sc * 16 + tl
        base = tl * rpt

        zb16 = jnp.zeros((32,), jnp.bfloat16)

        def _zb(i, _):
            zv[i, pl.ds(0, 32)] = zb16
            return _

        lax.fori_loop(0, zr, _zb, None)
        for z in range(zn):
            pltpu.sync_copy(zv, acc.at[pl.ds(base + z * zr, zr)])

        for c in range(k):
            plsc.subcore_barrier()
            tab = tables[c]

            def chunk(q, _):
                pltpu.sync_copy(src3d.at[wid, pl.ds(q * g, g)], src_c)
                pltpu.sync_copy(dst3d.at[wid, pl.ds(q * g, g)], dst_c)
                gd, sd = {}, {}
                for j in range(g):
                    b = j % nb
                    if j >= nb:
                        sd[j - nb].wait()
                    gd[j] = pltpu.async_copy(tab.at[src_c.at[j]], bufs[b],
                                             gsems[b])
                    if j >= 1:
                        jj = j - 1
                        gd[jj].wait()
                        sd[jj] = pltpu.async_copy(
                            bufs[jj % nb], acc.at[dst_c.at[jj]],
                            ssems[jj % nb], add=True)
                gd[g - 1].wait()
                sd[g - 1] = pltpu.async_copy(
                    bufs[(g - 1) % nb], acc.at[dst_c.at[g - 1]],
                    ssems[(g - 1) % nb], add=True)
                for jj in range(g - nb, g):
                    sd[jj].wait()
                return _

            lax.fori_loop(0, nch, chunk, None)
            plsc.subcore_barrier()
            pltpu.sync_copy(acc.at[pl.ds(base, rpt)],
                            out.at[sc, pl.ds(base, rpt),
                                   pl.ds(32 * c, 32)])
            if c + 1 < k:
                for z in range(zn):
                    pltpu.sync_copy(zv, acc.at[pl.ds(base + z * zr, zr)])

    return agg


@functools.lru_cache(maxsize=None)
def _make_counts(cfgs):
    out_npads = {}
    for _, np_, _, oi, _ in cfgs:
        out_npads[oi] = np_
    n_out = len(out_npads)
    max_np = max(c[1] for c in cfgs)
    max_g = max(c[2] for c in cfgs)
    rpt_max = max_np // 16
    zr = rpt_max // 8

    @functools.partial(
        pl.kernel, mesh=_sc_mesh(), compiler_params=_SC_PARAMS,
        out_type=tuple(jax.ShapeDtypeStruct((2, out_npads[i], 128),
                                            jnp.float32)
                       for i in range(n_out)),
        scratch_types=[
            pltpu.VMEM((max_g, 128), jnp.int32),
            pltpu.VMEM((128, 16), jnp.float32),
            pltpu.VMEM((zr, 16), jnp.float32),
            pltpu.VMEM_SHARED((max_np, 16), jnp.float32),
            pltpu.SemaphoreType.DMA,
        ],
    )
    def counts(*args):
        n = len(cfgs)
        dsts = args[:n]
        outs = args[n:n + n_out]
        dst_c, ones, zv, acc, sem = args[n + n_out:]
        sc = lax.axis_index("c")
        tl = lax.axis_index("s")
        wid = sc * 16 + tl

        _zero_vmem(zv, zr, 16)
        one = jnp.full((16,), 1.0 / 16.0, jnp.float32)

        def ob(i, _):
            ones[i, pl.ds(0, 16)] = one
            return _

        lax.fori_loop(0, 128, ob, None)

        for r, (e_pad, np_, g, oi, col) in enumerate(cfgs):
            ns = e_pad // 32 // 128
            nch = ns // g
            rpt = np_ // 16
            for z in range(8):
                pltpu.sync_copy(zv, acc.at[pl.ds(tl * rpt_max + z * zr, zr)])
            plsc.subcore_barrier()

            def chunk(q, _):
                pltpu.sync_copy(dsts[r].at[wid, pl.ds(q * g, g)],
                                dst_c.at[pl.ds(0, g)])
                sd = [pltpu.async_copy(ones, acc.at[dst_c.at[j]], sem,
                                       add=True) for j in range(g)]
                for d in sd:
                    d.wait()
                return _

            lax.fori_loop(0, nch, chunk, None)
            plsc.subcore_barrier()
            pltpu.sync_copy(acc.at[pl.ds(tl * rpt, rpt)],
                            outs[oi].at[sc, pl.ds(tl * rpt, rpt),
                                        pl.ds(16 * col, 16)])
            if r + 1 < n:
                plsc.subcore_barrier()

    return counts


def _pad_edges(e, e_pad, n_dst):
    pad = e_pad - e.shape[1]
    src = jnp.concatenate([e[0], jnp.zeros((pad,), jnp.int32)])
    dst = jnp.concatenate([e[1], jnp.full((pad,), n_dst, jnp.int32)])
    return src.reshape(32, -1, 128), dst.reshape(32, -1, 128)


def kernel(x_fm, x_sm, gf, period_vol, edge_qoq, edge_bel, edge_cp, edge_cd,
           edge_rev, params):
    p = params
    pe, wr_sum, bl_sum = _prep(
        p["W_pe"], period_vol, p["Wr_qoq"], p["Wr_cp"], p["Wr_rev"],
        p["bl_qoq"].reshape(1, H), p["bl_cp"].reshape(1, H),
        p["bl_rev"].reshape(1, H))

    xc = tuple(x_fm[:, 32 * c:32 * (c + 1)].astype(jnp.bfloat16)
               for c in range(4))
    pe_bf = pe.astype(jnp.bfloat16)
    edges = {"qoq": edge_qoq, "bel": edge_bel, "cp": edge_cp,
             "cd": edge_cd, "rev": edge_rev}
    ndst = {"qoq": N_FM, "bel": N_TP, "cp": N_FM, "cd": N_SM, "rev": N_FM}
    srcs, dsts = {}, {}
    for r, (e_pad, n_pad, k, g) in _CFG.items():
        srcs[r], dsts[r] = _pad_edges(edges[r], e_pad, ndst[r])

    rels = ["qoq", "bel", "cp", "cd", "rev"]
    slot = {"qoq": (0, 0), "bel": (1, 0), "cp": (0, 1), "cd": (2, 0),
            "rev": (0, 2)}
    cnt_cfg = tuple((_CFG[r][0], _CFG[r][1], _CFG[r][3]) + slot[r]
                    for r in rels)
    cfm, ctp, csm = _make_counts(cnt_cfg)(*[dsts[r] for r in rels])

    sd = _make_agg(*_CFG["cd"])(srcs["cd"], dsts["cd"], pe_bf)
    sb = _make_agg(*_CFG["bel"])(srcs["bel"], dsts["bel"], *xc)
    scp = _make_agg(*_CFG["cp"])(srcs["cp"], dsts["cp"], pe_bf)
    srv = _make_agg(*_CFG["rev"])(srcs["rev"], dsts["rev"], pe_bf)
    sq = _make_agg(*_CFG["qoq"])(srcs["qoq"], dsts["qoq"], *xc)

    fm = _fm_mean(x_fm, sq, cfm, scp, srv, p, wr_sum, bl_sum)
    tp = _tp_mean(pe, sb, ctp, p)
    sm = _sm_mean(x_sm, sd, csm, p)
    out = _head(fm, tp, sm, gf, p)
    return out.reshape(())
